# baseline (speedup 1.0000x reference)
import functools

import jax
import jax.numpy as jnp
from jax import lax
from jax.experimental import pallas as pl
from jax.experimental.pallas import tpu as pltpu

N = 10000
E = 160000
ET = E + N
B = 8
IN_DIM = 128
H1 = 64
H2 = 32
HEADS = 4
K_POWER = 10
ALPHA = 0.1


def _pool_final_body(z_ref, aux4_ref, batch_ref, gbc_ref,
                     wgz1_ref, wga1_ref, bg11_ref, wg12_ref, bg12_ref,
                     wnz1_ref, wna1_ref, bn11_ref, wn12_ref, bn12_ref,
                     wgz2_ref, wga2_ref, bg21_ref, wg22_ref, bg22_ref,
                     wnz2_ref, wna2_ref, bn21_ref, wn22_ref, bn22_ref,
                     wgz3_ref, wga3_ref, bg31_ref, wg32_ref, bg32_ref,
                     wnz3_ref, wna3_ref, bn31_ref, wn32_ref, bn32_ref,
                     w3z_ref, w3g_ref, w3a_ref, w3b_ref, b3_ref,
                     w4_ref, b4_ref, w5_ref, b5_ref,
                     out_ref):
    z = z_ref[...]
    aux4 = aux4_ref[...]
    aux3 = aux4[:, 0:3]
    batch = batch_ref[...]
    iota8 = lax.broadcasted_iota(jnp.int32, (1, B), 1)
    mask = (batch == iota8).astype(jnp.float32)

    def lrelu(t):
        return jnp.where(t >= 0, t, 0.2 * t)

    def pool(wgz, wga, bg1, wg2, bg2, wnz, wna, bn1, wn2, bn2):
        gate = jnp.maximum(z @ wgz + aux3 @ wga + bg1, 0.0) @ wg2 + bg2
        v = jnp.maximum(z @ wnz + aux3 @ wna + bn1, 0.0) @ wn2 + bn2
        gm = jnp.where(mask > 0, gate, -1e30)
        m = jnp.max(gm, axis=0)
        m = jnp.where(m > -1e29, m, 0.0)
        ex = jnp.exp(gate - mask @ m[:, None])
        den = jax.lax.dot_general(mask, ex, (((0,), (0,)), ((), ())))
        a = ex / (mask @ den + 1e-16)
        return jax.lax.dot_general(mask, a * v, (((0,), (0,)), ((), ())))

    g1 = pool(wgz1_ref[...], wga1_ref[...], bg11_ref[...], wg12_ref[...],
              bg12_ref[...], wnz1_ref[...], wna1_ref[...], bn11_ref[...],
              wn12_ref[...], bn12_ref[...])
    g2 = pool(wgz2_ref[...], wga2_ref[...], bg21_ref[...], wg22_ref[...],
              bg22_ref[...], wnz2_ref[...], wna2_ref[...], bn21_ref[...],
              wn22_ref[...], bn22_ref[...])
    g3 = pool(wgz3_ref[...], wga3_ref[...], bg31_ref[...], wg32_ref[...],
              bg32_ref[...], wnz3_ref[...], wna3_ref[...], bn31_ref[...],
              wn32_ref[...], bn32_ref[...])
    g = jnp.concatenate([g1, g2, g3], axis=1)
    tg = g @ w3g_ref[...] + gbc_ref[...] @ w3b_ref[...]
    s = lrelu(z @ w3z_ref[...] + aux4 @ w3a_ref[...] + mask @ tg + b3_ref[...])
    s = lrelu(s @ w4_ref[...] + b4_ref[...])
    s = s @ w5_ref[...] + b5_ref[...]
    s = jax.nn.sigmoid(s)
    out_ref[...] = jax.lax.dot_general(mask, s, (((0,), (0,)), ((), ())))


def _pool_final(z, aux4, batch, gbc, pool_ws, w3z, w3g, w3a, w3b, b3,
                w4, b4, w5, b5):
    flat = []
    for ws in pool_ws:
        flat.extend(ws)
    return pl.pallas_call(
        _pool_final_body,
        out_shape=jax.ShapeDtypeStruct((B, 1), jnp.float32),
    )(z, aux4, batch, gbc, *flat, w3z, w3g, w3a, w3b, b3, w4, b4, w5, b5)


def _seg_softmax_j(e, seg, num):
    m = jax.ops.segment_max(e, seg, num_segments=num)
    m = jnp.where(jnp.isfinite(m), m, 0.0)
    ex = jnp.exp(e - m[seg])
    den = jax.ops.segment_sum(ex, seg, num_segments=num)
    return ex / (den[seg] + 1e-16)


def _gat_j(h_in, src, dst, W, a_s, a_d, b, heads, ch, n):
    h = (h_in @ W).reshape(n, heads, ch)
    es = jnp.sum(h * a_s[None, :, :], axis=-1)
    ed = jnp.sum(h * a_d[None, :, :], axis=-1)
    e = jax.nn.leaky_relu(es[src] + ed[dst], 0.2)
    a = _seg_softmax_j(e, dst, n)
    out = jax.ops.segment_sum(h[src] * a[:, :, None], dst, num_segments=n)
    return out.reshape(n, heads * ch) + b


def kernel(x, edge_index, batch, J, saved_nodes, infected_nodes,
           size_connected, Omegas, Phis, Lambdas,
           W_gat1, att_src1, att_dst1, b_gat1, W_lin1, b_lin1,
           W_gat2, att_src2, att_dst2, b_gat2, W_lin2, b_lin2,
           Wg11, bg11, Wg12, bg12, Wn11, bn11, Wn12, bn12,
           Wg21, bg21, Wg22, bg22, Wn21, bn21, Wn22, bn22,
           Wg31, bg31, Wg32, bg32, Wn31, bn31, Wn32, bn32,
           W_lin3, b_lin3, W_lin4, b_lin4, W_lin5, b_lin5):
    n = N
    loop = jnp.arange(n, dtype=edge_index.dtype)
    src = jnp.concatenate([edge_index[0], loop])
    dst = jnp.concatenate([edge_index[1], loop])
    order = jnp.argsort(dst)
    src_s = src[order]
    dst_s = dst[order]

    h0 = jnp.concatenate([x, J, size_connected], axis=1)
    h = _gat_j(h0, src_s, dst_s, W_gat1, att_src1, att_dst1, b_gat1,
               HEADS, H1, n)
    h = jax.nn.leaky_relu(h @ W_lin1 + b_lin1, 0.2)
    h = _gat_j(h, src_s, dst_s, W_gat2, att_src2, att_dst2, b_gat2,
               HEADS, H2, n)
    h = jax.nn.leaky_relu(h @ W_lin2 + b_lin2, 0.2)

    deg = jax.ops.segment_sum(jnp.ones(src_s.shape[0], dtype=h.dtype),
                              dst_s, num_segments=n)
    dis = jnp.where(deg > 0, jax.lax.rsqrt(jnp.maximum(deg, 1e-12)), 0.0)
    norm = dis[src_s] * dis[dst_s]
    z = h
    for _ in range(K_POWER):
        z = ((1.0 - ALPHA)
             * jax.ops.segment_sum(z[src_s] * norm[:, None], dst_s,
                                   num_segments=n) + ALPHA * h)

    aux4 = jnp.concatenate([J, saved_nodes, infected_nodes, size_connected],
                           axis=1)
    gbc = jnp.concatenate([Omegas, Phis, Lambdas], axis=1)
    pool_ws = [
        (Wg11[:H2], Wg11[H2:], bg11, Wg12, bg12,
         Wn11[:H2], Wn11[H2:], bn11, Wn12, bn12),
        (Wg21[:H2], Wg21[H2:], bg21, Wg22, bg22,
         Wn21[:H2], Wn21[H2:], bn21, Wn22, bn22),
        (Wg31[:H2], Wg31[H2:], bg31, Wg32, bg32,
         Wn31[:H2], Wn31[H2:], bn31, Wn32, bn32),
    ]
    w3z = W_lin3[0:H2]
    w3g = W_lin3[H2:H2 + 3 * H2]
    w3a = W_lin3[H2 + 3 * H2:H2 + 3 * H2 + 4]
    w3b = W_lin3[H2 + 3 * H2 + 4:]
    return _pool_final(z, aux4, batch[:, None], gbc, pool_ws,
                       w3z, w3g, w3a, w3b, b_lin3,
                       W_lin4, b_lin4, W_lin5, b_lin5)


# speedup vs baseline: 13.8661x; 13.8661x over previous
import functools

import jax
import jax.numpy as jnp
from jax import lax
from jax.experimental import pallas as pl
from jax.experimental.pallas import tpu as pltpu
from jax.experimental.pallas import tpu_sc as plsc

N = 10000
E = 160000
B = 8
IN_DIM = 128
H1 = 64
H2 = 32
HEADS = 4
K_POWER = 10
ALPHA = 0.1

NP_ = 10240
TILES = 32
ROWS = NP_ // TILES
EP = 171264
CG = 32
CA = 128
BLK = 1024


def _lrelu(t):
    return jnp.where(t >= 0, t, 0.2 * t)


def _dense1_body(x_ref, aux_ref, wx_ref, wjs_ref, asd_ref, hrow_ref, ed_ref):
    h = x_ref[...] @ wx_ref[...] + aux_ref[...] @ wjs_ref[...]
    esed = h @ asd_ref[...]
    zpad = jnp.zeros((h.shape[0], 12), jnp.float32)
    hrow_ref[...] = jnp.concatenate([h, esed[:, 0:4], zpad], axis=1)
    ed_ref[...] = jnp.concatenate([esed[:, 4:8], zpad], axis=1)


def _dense1(x_p, aux_p, wx, wjs, asd):
    g = NP_ // BLK
    return pl.pallas_call(
        _dense1_body,
        grid=(g,),
        in_specs=[
            pl.BlockSpec((BLK, IN_DIM), lambda i: (i, 0)),
            pl.BlockSpec((BLK, 8), lambda i: (i, 0)),
            pl.BlockSpec(wx.shape, lambda i: (0, 0)),
            pl.BlockSpec(wjs.shape, lambda i: (0, 0)),
            pl.BlockSpec(asd.shape, lambda i: (0, 0)),
        ],
        out_specs=[
            pl.BlockSpec((BLK, 272), lambda i: (i, 0)),
            pl.BlockSpec((BLK, 16), lambda i: (i, 0)),
        ],
        out_shape=[
            jax.ShapeDtypeStruct((NP_, 272), jnp.float32),
            jax.ShapeDtypeStruct((NP_, 16), jnp.float32),
        ],
    )(x_p, aux_p, wx, wjs, asd)


def _dense2_body(o_ref, bg_ref, wl_ref, bl_ref, wg_ref, asd_ref,
                 hrow_ref, ed_ref):
    t = _lrelu((o_ref[...] + bg_ref[...]) @ wl_ref[...] + bl_ref[...])
    h2 = t @ wg_ref[...]
    esed = h2 @ asd_ref[...]
    zpad = jnp.zeros((h2.shape[0], 12), jnp.float32)
    hrow_ref[...] = jnp.concatenate([h2, esed[:, 0:4], zpad], axis=1)
    ed_ref[...] = jnp.concatenate([esed[:, 4:8], zpad], axis=1)


def _dense2(out1, b_gat1, W_lin1, b_lin1, W_gat2, asd2):
    g = NP_ // BLK
    return pl.pallas_call(
        _dense2_body,
        grid=(g,),
        in_specs=[
            pl.BlockSpec((BLK, 256), lambda i: (i, 0)),
            pl.BlockSpec((1, 256), lambda i: (0, 0)),
            pl.BlockSpec(W_lin1.shape, lambda i: (0, 0)),
            pl.BlockSpec((1, H1), lambda i: (0, 0)),
            pl.BlockSpec(W_gat2.shape, lambda i: (0, 0)),
            pl.BlockSpec(asd2.shape, lambda i: (0, 0)),
        ],
        out_specs=[
            pl.BlockSpec((BLK, 144), lambda i: (i, 0)),
            pl.BlockSpec((BLK, 16), lambda i: (i, 0)),
        ],
        out_shape=[
            jax.ShapeDtypeStruct((NP_, 144), jnp.float32),
            jax.ShapeDtypeStruct((NP_, 16), jnp.float32),
        ],
    )(out1, b_gat1[None, :], W_lin1, b_lin1[None, :], W_gat2, asd2)


def _dense3_body(o_ref, deg_ref, bg_ref, wl_ref, bl_ref,
                 h_ref, dis_ref, y_ref):
    h32 = _lrelu((o_ref[...] + bg_ref[...]) @ wl_ref[...] + bl_ref[...])
    deg = deg_ref[...]
    dis = jnp.where(deg > 0, lax.rsqrt(jnp.maximum(deg, 1e-12)), 0.0)
    h_ref[...] = h32
    dis_ref[...] = jnp.concatenate(
        [dis, jnp.zeros((dis.shape[0], 15), jnp.float32)], axis=1)
    y_ref[...] = h32 * dis


def _dense3(out2, deg, b_gat2, W_lin2, b_lin2):
    g = NP_ // BLK
    return pl.pallas_call(
        _dense3_body,
        grid=(g,),
        in_specs=[
            pl.BlockSpec((BLK, 128), lambda i: (i, 0)),
            pl.BlockSpec((BLK, 1), lambda i: (i, 0)),
            pl.BlockSpec((1, 128), lambda i: (0, 0)),
            pl.BlockSpec(W_lin2.shape, lambda i: (0, 0)),
            pl.BlockSpec((1, H2), lambda i: (0, 0)),
        ],
        out_specs=[
            pl.BlockSpec((BLK, H2), lambda i: (i, 0)),
            pl.BlockSpec((BLK, 16), lambda i: (i, 0)),
            pl.BlockSpec((BLK, H2), lambda i: (i, 0)),
        ],
        out_shape=[
            jax.ShapeDtypeStruct((NP_, H2), jnp.float32),
            jax.ShapeDtypeStruct((NP_, 16), jnp.float32),
            jax.ShapeDtypeStruct((NP_, H2), jnp.float32),
        ],
    )(out2, deg, b_gat2[None, :], W_lin2, b_lin2[None, :])


def _bcast_lane(v, i):
    return jnp.broadcast_to(v[i], (16,))


def _make_gat_sc(ch):
    roww = ch + 16
    cph = ch // HEADS
    sub = cph // 16
    nacc = HEADS * sub
    mesh = plsc.VectorSubcoreMesh(core_axis_name="c", subcore_axis_name="s")

    @functools.partial(
        pl.kernel,
        mesh=mesh,
        out_type=jax.ShapeDtypeStruct((NP_ * ch,), jnp.float32),
        compiler_params=pltpu.CompilerParams(use_tc_tiling_on_sc=False),
        scratch_types=[
            pltpu.VMEM((TILES * 16,), jnp.int32),
            pltpu.VMEM((ROWS * 16,), jnp.float32),
            pltpu.VMEM((CG,), jnp.int32),
            pltpu.VMEM((CG,), jnp.int32),
            pltpu.VMEM((CG, roww), jnp.float32),
            pltpu.VMEM((ROWS * ch,), jnp.float32),
            pltpu.SemaphoreType.DMA,
        ],
    )
    def gat_sc(hrow_hbm, edp_hbm, srcp_hbm, dstp_hbm, tptr_hbm, out_hbm,
               tptr_v, ed_v, idx_v, dst_v, hbuf_v, out_v, sem):
        wid = lax.axis_index("s") * 2 + lax.axis_index("c")
        d0 = wid * ROWS
        pltpu.sync_copy(tptr_hbm, tptr_v)
        pltpu.sync_copy(edp_hbm.at[pl.ds(d0 * 16, ROWS * 16)], ed_v)
        erow = tptr_v[pl.ds(wid * 16, 16)]
        e0 = erow[0]
        e1 = erow[1]
        base0 = (e0 // CG) * CG
        nch = (e1 - base0 + CG - 1) // CG

        def write_row(cur, den, accs):
            inv = 1.0 / (den + 1e-16)
            for h in range(HEADS):
                invh = _bcast_lane(inv, h)
                for j in range(sub):
                    out_v[pl.ds(cur * ch + h * cph + j * 16, 16)] = (
                        accs[h * sub + j] * invh)

        def chunk_body(c, carry):
            cur, den, accs = carry
            base = base0 + c * CG
            pltpu.sync_copy(srcp_hbm.at[pl.ds(base, CG)], idx_v)
            pltpu.sync_copy(dstp_hbm.at[pl.ds(base, CG)], dst_v)
            pltpu.async_copy(hrow_hbm.at[idx_v], hbuf_v, sem).wait()
            for i in range(CG):
                if i % 16 == 0:
                    dvec = dst_v[pl.ds(i, 16)]
                eidx = base + i
                valid = jnp.logical_and(eidx >= e0, eidx < e1)
                dl = jnp.where(valid, dvec[i % 16] - d0, cur)
                flush = dl != cur

                def do_flush(den_, accs_):
                    write_row(cur, den_, accs_)
                    return (jnp.zeros((16,), jnp.float32),
                            tuple(jnp.zeros((16,), jnp.float32)
                                  for _ in range(nacc)))

                def keep(den_, accs_):
                    return (den_, accs_)

                den, accs = lax.cond(flush, do_flush, keep, den, accs)
                esv = hbuf_v[i, pl.ds(ch, 16)]
                edv = ed_v[pl.ds(dl * 16, 16)]
                e16 = _lrelu(esv + edv)
                ex = jnp.where(valid, jnp.exp(e16),
                               jnp.zeros((16,), jnp.float32))
                den = den + ex
                accs = list(accs)
                for h in range(HEADS):
                    w = _bcast_lane(ex, h)
                    for j in range(sub):
                        k = h * sub + j
                        accs[k] = accs[k] + w * hbuf_v[
                            i, pl.ds(h * cph + j * 16, 16)]
                accs = tuple(accs)
                cur = dl
            return (cur, den, accs)

        init = (jnp.int32(0), jnp.zeros((16,), jnp.float32),
                tuple(jnp.zeros((16,), jnp.float32) for _ in range(nacc)))
        cur, den, accs = lax.fori_loop(0, nch, chunk_body, init)
        write_row(cur, den, accs)
        pltpu.sync_copy(out_v, out_hbm.at[pl.ds(d0 * ch, ROWS * ch)])

    return gat_sc


_gat_sc_256 = _make_gat_sc(256)
_gat_sc_128 = _make_gat_sc(128)


def _make_appnp_sc():
    mesh = plsc.VectorSubcoreMesh(core_axis_name="c", subcore_axis_name="s")

    @functools.partial(
        pl.kernel,
        mesh=mesh,
        out_type=[
            jax.ShapeDtypeStruct((NP_ * H2,), jnp.float32),
            jax.ShapeDtypeStruct((NP_ * H2,), jnp.float32),
        ],
        compiler_params=pltpu.CompilerParams(use_tc_tiling_on_sc=False),
        scratch_types=[
            pltpu.VMEM((TILES * 16,), jnp.int32),
            pltpu.VMEM((ROWS * H2,), jnp.float32),
            pltpu.VMEM((ROWS * 16,), jnp.float32),
            pltpu.VMEM((CA,), jnp.int32),
            pltpu.VMEM((CA,), jnp.int32),
            pltpu.VMEM((CA, H2), jnp.float32),
            pltpu.VMEM((ROWS * H2,), jnp.float32),
            pltpu.VMEM((ROWS * H2,), jnp.float32),
            pltpu.SemaphoreType.DMA,
        ],
    )
    def appnp_sc(y_hbm, h_hbm, dis_hbm, srcp_hbm, dstp_hbm, tptr_hbm,
                 z_hbm, yo_hbm,
                 tptr_v, h_v, dis_v, idx_v, dst_v, ybuf_v, z_v, yo_v, sem):
        wid = lax.axis_index("s") * 2 + lax.axis_index("c")
        d0 = wid * ROWS
        pltpu.sync_copy(tptr_hbm, tptr_v)
        pltpu.sync_copy(h_hbm.at[pl.ds(d0 * H2, ROWS * H2)], h_v)
        pltpu.sync_copy(dis_hbm.at[pl.ds(d0 * 16, ROWS * 16)], dis_v)
        erow = tptr_v[pl.ds(wid * 16, 16)]
        e0 = erow[0]
        e1 = erow[1]
        base0 = (e0 // CA) * CA
        nch = (e1 - base0 + CA - 1) // CA

        def write_row(cur, a0, a1):
            dvec = dis_v[pl.ds(cur * 16, 16)]
            sv = jnp.broadcast_to(dvec[0], (16,))
            z0 = (0.9 * sv) * a0 + 0.1 * h_v[pl.ds(cur * H2, 16)]
            z1 = (0.9 * sv) * a1 + 0.1 * h_v[pl.ds(cur * H2 + 16, 16)]
            z_v[pl.ds(cur * H2, 16)] = z0
            z_v[pl.ds(cur * H2 + 16, 16)] = z1
            yo_v[pl.ds(cur * H2, 16)] = sv * z0
            yo_v[pl.ds(cur * H2 + 16, 16)] = sv * z1

        def chunk_body(c, carry):
            cur, a0, a1 = carry
            base = base0 + c * CA
            pltpu.sync_copy(srcp_hbm.at[pl.ds(base, CA)], idx_v)
            pltpu.sync_copy(dstp_hbm.at[pl.ds(base, CA)], dst_v)
            pltpu.async_copy(y_hbm.at[idx_v], ybuf_v, sem).wait()
            for i in range(CA):
                if i % 16 == 0:
                    dvec = dst_v[pl.ds(i, 16)]
                eidx = base + i
                valid = jnp.logical_and(eidx >= e0, eidx < e1)
                dl = jnp.where(valid, dvec[i % 16] - d0, cur)
                flush = dl != cur

                def do_flush(a0_, a1_):
                    write_row(cur, a0_, a1_)
                    return (jnp.zeros((16,), jnp.float32),
                            jnp.zeros((16,), jnp.float32))

                def keep(a0_, a1_):
                    return (a0_, a1_)

                a0, a1 = lax.cond(flush, do_flush, keep, a0, a1)
                vm = jnp.where(valid, jnp.float32(1.0), jnp.float32(0.0))
                vmv = jnp.broadcast_to(vm, (16,))
                a0 = a0 + vmv * ybuf_v[i, pl.ds(0, 16)]
                a1 = a1 + vmv * ybuf_v[i, pl.ds(16, 16)]
                cur = dl
            return (cur, a0, a1)

        init = (jnp.int32(0), jnp.zeros((16,), jnp.float32),
                jnp.zeros((16,), jnp.float32))
        cur, a0, a1 = lax.fori_loop(0, nch, chunk_body, init)
        write_row(cur, a0, a1)
        pltpu.sync_copy(z_v, z_hbm.at[pl.ds(d0 * H2, ROWS * H2)])
        pltpu.sync_copy(yo_v, yo_hbm.at[pl.ds(d0 * H2, ROWS * H2)])

    return appnp_sc


_appnp_sc = _make_appnp_sc()


def _pool_final_body(z_ref, aux4_ref, batch_ref, gbc_ref,
                     wgz1_ref, wga1_ref, bg11_ref, wg12_ref, bg12_ref,
                     wnz1_ref, wna1_ref, bn11_ref, wn12_ref, bn12_ref,
                     wgz2_ref, wga2_ref, bg21_ref, wg22_ref, bg22_ref,
                     wnz2_ref, wna2_ref, bn21_ref, wn22_ref, bn22_ref,
                     wgz3_ref, wga3_ref, bg31_ref, wg32_ref, bg32_ref,
                     wnz3_ref, wna3_ref, bn31_ref, wn32_ref, bn32_ref,
                     w3z_ref, w3g_ref, w3a_ref, w3b_ref, b3_ref,
                     w4_ref, b4_ref, w5_ref, b5_ref,
                     out_ref):
    z = z_ref[...]
    aux4 = aux4_ref[...]
    aux3 = aux4[:, 0:3]
    batch = batch_ref[...]
    iota8 = lax.broadcasted_iota(jnp.int32, (1, B), 1)
    mask = (batch == iota8).astype(jnp.float32)

    def pool(wgz, wga, bg1, wg2, bg2, wnz, wna, bn1, wn2, bn2):
        gate = jnp.maximum(z @ wgz + aux3 @ wga + bg1, 0.0) @ wg2 + bg2
        v = jnp.maximum(z @ wnz + aux3 @ wna + bn1, 0.0) @ wn2 + bn2
        gm = jnp.where(mask > 0, gate, -1e30)
        m = jnp.max(gm, axis=0)
        m = jnp.where(m > -1e29, m, 0.0)
        ex = jnp.exp(gate - mask @ m[:, None])
        den = lax.dot_general(mask, ex, (((0,), (0,)), ((), ())))
        a = ex / (mask @ den + 1e-16)
        return lax.dot_general(mask, a * v, (((0,), (0,)), ((), ())))

    g1 = pool(wgz1_ref[...], wga1_ref[...], bg11_ref[...], wg12_ref[...],
              bg12_ref[...], wnz1_ref[...], wna1_ref[...], bn11_ref[...],
              wn12_ref[...], bn12_ref[...])
    g2 = pool(wgz2_ref[...], wga2_ref[...], bg21_ref[...], wg22_ref[...],
              bg22_ref[...], wnz2_ref[...], wna2_ref[...], bn21_ref[...],
              wn22_ref[...], bn22_ref[...])
    g3 = pool(wgz3_ref[...], wga3_ref[...], bg31_ref[...], wg32_ref[...],
              bg32_ref[...], wnz3_ref[...], wna3_ref[...], bn31_ref[...],
              wn32_ref[...], bn32_ref[...])
    g = jnp.concatenate([g1, g2, g3], axis=1)
    tg = g @ w3g_ref[...] + gbc_ref[...] @ w3b_ref[...]
    s = _lrelu(z @ w3z_ref[...] + aux4 @ w3a_ref[...] + mask @ tg
               + b3_ref[...])
    s = _lrelu(s @ w4_ref[...] + b4_ref[...])
    s = s @ w5_ref[...] + b5_ref[...]
    s = jax.nn.sigmoid(s)
    out_ref[...] = lax.dot_general(mask, s, (((0,), (0,)), ((), ())))


def _pool_final(z, aux4, batch, gbc, pool_ws, w3z, w3g, w3a, w3b, b3,
                w4, b4, w5, b5):
    flat = []
    for ws in pool_ws:
        flat.extend(ws)
    return pl.pallas_call(
        _pool_final_body,
        out_shape=jax.ShapeDtypeStruct((B, 1), jnp.float32),
    )(z, aux4, batch, gbc, *flat, w3z, w3g, w3a, w3b, b3, w4, b4, w5, b5)


def _block_diag_att(att_src, att_dst):
    heads, ch = att_src.shape
    eye = jnp.eye(heads, dtype=att_src.dtype)
    a_s = jnp.einsum("hc,hk->hck", att_src, eye).reshape(heads * ch, heads)
    a_d = jnp.einsum("hc,hk->hck", att_dst, eye).reshape(heads * ch, heads)
    return jnp.concatenate([a_s, a_d], axis=1)


def kernel(x, edge_index, batch, J, saved_nodes, infected_nodes,
           size_connected, Omegas, Phis, Lambdas,
           W_gat1, att_src1, att_dst1, b_gat1, W_lin1, b_lin1,
           W_gat2, att_src2, att_dst2, b_gat2, W_lin2, b_lin2,
           Wg11, bg11, Wg12, bg12, Wn11, bn11, Wn12, bn12,
           Wg21, bg21, Wg22, bg22, Wn21, bn21, Wn22, bn22,
           Wg31, bg31, Wg32, bg32, Wn31, bn31, Wn32, bn32,
           W_lin3, b_lin3, W_lin4, b_lin4, W_lin5, b_lin5):
    idt = edge_index.dtype
    loop = jnp.arange(N, dtype=idt)
    src = jnp.concatenate([edge_index[0], loop])
    dst = jnp.concatenate([edge_index[1], loop])
    dst_s, src_s = lax.sort((dst, src), num_keys=1)

    fake_dst = jnp.arange(N, NP_, dtype=idt)
    dst_f = jnp.concatenate([dst_s, fake_dst])
    src_f = jnp.concatenate([src_s, jnp.zeros((NP_ - N,), idt)])
    npad = EP - dst_f.shape[0]
    dstp = jnp.concatenate([dst_f, jnp.zeros((npad,), idt)])
    srcp = jnp.concatenate([src_f, jnp.zeros((npad,), idt)])

    bounds = jnp.arange(TILES + 1, dtype=idt) * ROWS
    tp = jnp.searchsorted(dst_f, bounds).astype(jnp.int32)
    tptr = jnp.concatenate(
        [tp[:TILES, None], tp[1:, None],
         jnp.zeros((TILES, 14), jnp.int32)], axis=1).reshape(-1)
    rp = jnp.searchsorted(dst_s, jnp.arange(N + 1, dtype=idt))
    deg = (rp[1:] - rp[:-1]).astype(jnp.float32)[:, None]
    deg = jnp.concatenate([deg, jnp.zeros((NP_ - N, 1), jnp.float32)])

    zrows = jnp.zeros((NP_ - N, 1), jnp.float32)
    x_p = jnp.concatenate([x, jnp.zeros((NP_ - N, IN_DIM), jnp.float32)])
    aux_p = jnp.concatenate([
        jnp.concatenate([J, zrows]), jnp.concatenate([size_connected, zrows]),
        jnp.zeros((NP_, 6), jnp.float32)], axis=1)

    wx = W_gat1[:IN_DIM]
    wjs = jnp.concatenate(
        [W_gat1[IN_DIM:], jnp.zeros((6, HEADS * H1), jnp.float32)])
    asd1 = _block_diag_att(att_src1, att_dst1)
    asd2 = _block_diag_att(att_src2, att_dst2)

    hrow1, edp1 = _dense1(x_p, aux_p, wx, wjs, asd1)
    out1 = _gat_sc_256(hrow1, edp1.reshape(-1), srcp, dstp, tptr)
    hrow2, edp2 = _dense2(out1.reshape(NP_, 256), b_gat1, W_lin1, b_lin1,
                          W_gat2, asd2)
    out2 = _gat_sc_128(hrow2, edp2.reshape(-1), srcp, dstp, tptr)
    h32, dis16, y = _dense3(out2.reshape(NP_, 128), deg, b_gat2,
                            W_lin2, b_lin2)

    h32f = h32.reshape(-1)
    dis16f = dis16.reshape(-1)
    z = None
    for _ in range(K_POWER):
        zf, yf = _appnp_sc(y, h32f, dis16f, srcp, dstp, tptr)
        y = yf.reshape(NP_, H2)
        z = zf.reshape(NP_, H2)

    aux4 = jnp.concatenate([J, saved_nodes, infected_nodes, size_connected],
                           axis=1)
    gbc = jnp.concatenate([Omegas, Phis, Lambdas], axis=1)
    pool_ws = [
        (Wg11[:H2], Wg11[H2:], bg11, Wg12, bg12,
         Wn11[:H2], Wn11[H2:], bn11, Wn12, bn12),
        (Wg21[:H2], Wg21[H2:], bg21, Wg22, bg22,
         Wn21[:H2], Wn21[H2:], bn21, Wn22, bn22),
        (Wg31[:H2], Wg31[H2:], bg31, Wg32, bg32,
         Wn31[:H2], Wn31[H2:], bn31, Wn32, bn32),
    ]
    w3z = W_lin3[0:H2]
    w3g = W_lin3[H2:H2 + 3 * H2]
    w3a = W_lin3[H2 + 3 * H2:H2 + 3 * H2 + 4]
    w3b = W_lin3[H2 + 3 * H2 + 4:]
    return _pool_final(z[:N], aux4, batch[:, None], gbc, pool_ws,
                       w3z, w3g, w3a, w3b, b_lin3,
                       W_lin4, b_lin4, W_lin5, b_lin5)


# speedup vs baseline: 15.4545x; 1.1146x over previous
import functools

import jax
import jax.numpy as jnp
from jax import lax
from jax.experimental import pallas as pl
from jax.experimental.pallas import tpu as pltpu
from jax.experimental.pallas import tpu_sc as plsc

N = 10000
E = 160000
B = 8
IN_DIM = 128
H1 = 64
H2 = 32
HEADS = 4
K_POWER = 10
ALPHA = 0.1

NP_ = 10240
TILES = 32
ROWS = NP_ // TILES
ECAP = 6144
EP = 170240 + ECAP
CG = 64
CA = 128
BLK = 1024


def _lrelu(t):
    return jnp.where(t >= 0, t, 0.2 * t)


def _dense1_body(x_ref, aux_ref, wx_ref, wjs_ref, asd_ref, hrow_ref, ed_ref):
    h = x_ref[...] @ wx_ref[...] + aux_ref[...] @ wjs_ref[...]
    esed = h @ asd_ref[...]
    zpad = jnp.zeros((h.shape[0], 12), jnp.float32)
    hrow_ref[...] = jnp.concatenate([h, esed[:, 0:4], zpad], axis=1)
    ed_ref[...] = jnp.concatenate([esed[:, 4:8], zpad], axis=1)


def _dense1(x_p, aux_p, wx, wjs, asd):
    g = NP_ // BLK
    return pl.pallas_call(
        _dense1_body,
        grid=(g,),
        in_specs=[
            pl.BlockSpec((BLK, IN_DIM), lambda i: (i, 0)),
            pl.BlockSpec((BLK, 8), lambda i: (i, 0)),
            pl.BlockSpec(wx.shape, lambda i: (0, 0)),
            pl.BlockSpec(wjs.shape, lambda i: (0, 0)),
            pl.BlockSpec(asd.shape, lambda i: (0, 0)),
        ],
        out_specs=[
            pl.BlockSpec((BLK, 272), lambda i: (i, 0)),
            pl.BlockSpec((BLK, 16), lambda i: (i, 0)),
        ],
        out_shape=[
            jax.ShapeDtypeStruct((NP_, 272), jnp.float32),
            jax.ShapeDtypeStruct((NP_, 16), jnp.float32),
        ],
    )(x_p, aux_p, wx, wjs, asd)


def _dense2_body(o_ref, den_ref, r1_ref, bg_ref, wl_ref, bl_ref, wg_ref,
                 asd_ref, hrow_ref, ed_ref):
    inv4 = 1.0 / (den_ref[...][:, 0:4] + 1e-16)
    o = o_ref[...] * (inv4 @ r1_ref[...])
    t = _lrelu((o + bg_ref[...]) @ wl_ref[...] + bl_ref[...])
    h2 = t @ wg_ref[...]
    esed = h2 @ asd_ref[...]
    zpad = jnp.zeros((h2.shape[0], 12), jnp.float32)
    hrow_ref[...] = jnp.concatenate([h2, esed[:, 0:4], zpad], axis=1)
    ed_ref[...] = jnp.concatenate([esed[:, 4:8], zpad], axis=1)


def _dense2(out1, den1, r1, b_gat1, W_lin1, b_lin1, W_gat2, asd2):
    g = NP_ // BLK
    return pl.pallas_call(
        _dense2_body,
        grid=(g,),
        in_specs=[
            pl.BlockSpec((BLK, 256), lambda i: (i, 0)),
            pl.BlockSpec((BLK, 16), lambda i: (i, 0)),
            pl.BlockSpec((HEADS, 256), lambda i: (0, 0)),
            pl.BlockSpec((1, 256), lambda i: (0, 0)),
            pl.BlockSpec(W_lin1.shape, lambda i: (0, 0)),
            pl.BlockSpec((1, H1), lambda i: (0, 0)),
            pl.BlockSpec(W_gat2.shape, lambda i: (0, 0)),
            pl.BlockSpec(asd2.shape, lambda i: (0, 0)),
        ],
        out_specs=[
            pl.BlockSpec((BLK, 144), lambda i: (i, 0)),
            pl.BlockSpec((BLK, 16), lambda i: (i, 0)),
        ],
        out_shape=[
            jax.ShapeDtypeStruct((NP_, 144), jnp.float32),
            jax.ShapeDtypeStruct((NP_, 16), jnp.float32),
        ],
    )(out1, den1, r1, b_gat1[None, :], W_lin1, b_lin1[None, :], W_gat2, asd2)


def _dense3_body(o_ref, den_ref, r2_ref, deg_ref, bg_ref, wl_ref, bl_ref,
                 h_ref, dis_ref, y_ref):
    inv4 = 1.0 / (den_ref[...][:, 0:4] + 1e-16)
    o = o_ref[...] * (inv4 @ r2_ref[...])
    h32 = _lrelu((o + bg_ref[...]) @ wl_ref[...] + bl_ref[...])
    deg = deg_ref[...]
    dis = jnp.where(deg > 0, lax.rsqrt(jnp.maximum(deg, 1e-12)), 0.0)
    h_ref[...] = h32
    dis_ref[...] = jnp.concatenate(
        [dis, jnp.zeros((dis.shape[0], 15), jnp.float32)], axis=1)
    y_ref[...] = h32 * dis


def _dense3(out2, den2, r2, deg, b_gat2, W_lin2, b_lin2):
    g = NP_ // BLK
    return pl.pallas_call(
        _dense3_body,
        grid=(g,),
        in_specs=[
            pl.BlockSpec((BLK, 128), lambda i: (i, 0)),
            pl.BlockSpec((BLK, 16), lambda i: (i, 0)),
            pl.BlockSpec((HEADS, 128), lambda i: (0, 0)),
            pl.BlockSpec((BLK, 1), lambda i: (i, 0)),
            pl.BlockSpec((1, 128), lambda i: (0, 0)),
            pl.BlockSpec(W_lin2.shape, lambda i: (0, 0)),
            pl.BlockSpec((1, H2), lambda i: (0, 0)),
        ],
        out_specs=[
            pl.BlockSpec((BLK, H2), lambda i: (i, 0)),
            pl.BlockSpec((BLK, 16), lambda i: (i, 0)),
            pl.BlockSpec((BLK, H2), lambda i: (i, 0)),
        ],
        out_shape=[
            jax.ShapeDtypeStruct((NP_, H2), jnp.float32),
            jax.ShapeDtypeStruct((NP_, 16), jnp.float32),
            jax.ShapeDtypeStruct((NP_, H2), jnp.float32),
        ],
    )(out2, den2, r2, deg, b_gat2[None, :], W_lin2, b_lin2[None, :])


def _bcast_lane(v, i):
    return jnp.broadcast_to(v[i], (16,))


def _make_gat_sc(ch):
    roww = ch + 16
    cph = ch // HEADS
    sub = cph // 16
    nacc = HEADS * sub
    mesh = plsc.VectorSubcoreMesh(core_axis_name="c", subcore_axis_name="s")

    @functools.partial(
        pl.kernel,
        mesh=mesh,
        out_type=[
            jax.ShapeDtypeStruct((NP_ * ch,), jnp.float32),
            jax.ShapeDtypeStruct((NP_ * 16,), jnp.float32),
        ],
        compiler_params=pltpu.CompilerParams(use_tc_tiling_on_sc=False),
        scratch_types=[
            pltpu.VMEM(((ROWS + 16) * 16,), jnp.int32),
            pltpu.VMEM((ROWS * 16,), jnp.float32),
            pltpu.VMEM((ECAP,), jnp.int32),
            pltpu.VMEM((CG, roww), jnp.float32),
            pltpu.VMEM((ROWS * ch,), jnp.float32),
            pltpu.VMEM((ROWS * 16,), jnp.float32),
            pltpu.SemaphoreType.DMA,
        ],
    )
    def gat_sc(hrow_hbm, edp_hbm, rp_hbm, srcp_hbm, out_hbm, deno_hbm,
               rp_v, ed_v, srcall_v, gbuf_v, out_v, den_v, sem):
        zed = jnp.zeros((16,), jnp.float32)
        wid = lax.axis_index("s") * 2 + lax.axis_index("c")
        d0 = wid * ROWS
        pltpu.sync_copy(rp_hbm.at[pl.ds(d0 * 16, (ROWS + 16) * 16)], rp_v)
        pltpu.sync_copy(edp_hbm.at[pl.ds(d0 * 16, ROWS * 16)], ed_v)
        e0 = rp_v[pl.ds(0, 16)][0]
        e1 = rp_v[pl.ds(ROWS * 16, 16)][0]
        base0 = (e0 // CG) * CG
        nsuper = (e1 - base0 + ECAP - 1) // ECAP

        def per_edge(masked, gbase, gi, carry):
            r, row_end, den, accs = carry
            gidx = gbase + gi
            esv = gbuf_v[gi, pl.ds(ch, 16)]
            edrow = ed_v[pl.ds(r * 16, 16)]
            ex = jnp.exp(_lrelu(esv + edrow))
            if masked:
                vm = jnp.logical_and(gidx >= e0, gidx < e1)
                ex = jnp.where(vm, ex, zed)
            den = den + ex
            naccs = []
            for h in range(HEADS):
                w = _bcast_lane(ex, h)
                for j in range(sub):
                    naccs.append(accs[h * sub + j]
                                 + w * gbuf_v[gi, pl.ds(h * cph + j * 16, 16)])
            is_end = (gidx + 1) == row_end
            if masked:
                is_end = jnp.logical_and(is_end, vm)

            def flush(den_, accs_):
                den_v[pl.ds(r * 16, 16)] = den_
                for k in range(nacc):
                    out_v[pl.ds(r * ch + k * 16, 16)] = accs_[k]
                r2 = r + 1
                row_end2 = rp_v[pl.ds((r2 + 1) * 16, 16)][0]
                return (r2, row_end2, zed,
                        tuple(zed for _ in range(nacc)))

            def keep(den_, accs_):
                return (r, row_end, den_, accs_)

            return lax.cond(is_end, flush, keep, den, tuple(naccs))

        def super_body(s, carry):
            sb = base0 + s * ECAP
            pltpu.sync_copy(srcp_hbm.at[pl.ds(sb, ECAP)], srcall_v)
            cnt = jnp.minimum(ECAP, e1 - sb)
            nchs = (cnt + CG - 1) // CG

            def proc_chunk(carry, coff, masked):
                pltpu.async_copy(
                    hrow_hbm.at[srcall_v.at[pl.ds(coff, CG)]],
                    gbuf_v, sem).wait()
                gbase = sb + coff

                def group(g, cr):
                    for i in range(16):
                        cr = per_edge(masked, gbase, g * 16 + i, cr)
                    return cr

                return lax.fori_loop(0, CG // 16, group, carry)

            carry = proc_chunk(carry, 0, True)
            carry = lax.fori_loop(
                1, nchs - 1,
                lambda c, cr: proc_chunk(cr, c * CG, False), carry)
            carry = lax.fori_loop(
                jnp.maximum(nchs - 1, 1), nchs,
                lambda c, cr: proc_chunk(cr, c * CG, True), carry)
            return carry

        init = (jnp.int32(0), rp_v[pl.ds(16, 16)][0],
                zed, tuple(zed for _ in range(nacc)))
        lax.fori_loop(0, nsuper, super_body, init)
        pltpu.sync_copy(out_v, out_hbm.at[pl.ds(d0 * ch, ROWS * ch)])
        pltpu.sync_copy(den_v, deno_hbm.at[pl.ds(d0 * 16, ROWS * 16)])

    return gat_sc


_gat_sc_256 = _make_gat_sc(256)
_gat_sc_128 = _make_gat_sc(128)


def _make_appnp_sc():
    mesh = plsc.VectorSubcoreMesh(core_axis_name="c", subcore_axis_name="s")

    @functools.partial(
        pl.kernel,
        mesh=mesh,
        out_type=[
            jax.ShapeDtypeStruct((NP_ * H2,), jnp.float32),
            jax.ShapeDtypeStruct((NP_ * H2,), jnp.float32),
        ],
        compiler_params=pltpu.CompilerParams(use_tc_tiling_on_sc=False),
        scratch_types=[
            pltpu.VMEM(((ROWS + 16) * 16,), jnp.int32),
            pltpu.VMEM((ROWS * H2,), jnp.float32),
            pltpu.VMEM((ROWS * 16,), jnp.float32),
            pltpu.VMEM((ECAP,), jnp.int32),
            pltpu.VMEM((CA, H2), jnp.float32),
            pltpu.VMEM((ROWS * H2,), jnp.float32),
            pltpu.VMEM((ROWS * H2,), jnp.float32),
            pltpu.SemaphoreType.DMA,
        ],
    )
    def appnp_sc(y_hbm, h_hbm, dis_hbm, rp_hbm, srcp_hbm,
                 z_hbm, yo_hbm,
                 rp_v, h_v, dis_v, srcall_v, ybuf_v, z_v, yo_v, sem):
        zed = jnp.zeros((16,), jnp.float32)
        wid = lax.axis_index("s") * 2 + lax.axis_index("c")
        d0 = wid * ROWS
        pltpu.sync_copy(rp_hbm.at[pl.ds(d0 * 16, (ROWS + 16) * 16)], rp_v)
        pltpu.sync_copy(h_hbm.at[pl.ds(d0 * H2, ROWS * H2)], h_v)
        pltpu.sync_copy(dis_hbm.at[pl.ds(d0 * 16, ROWS * 16)], dis_v)
        e0 = rp_v[pl.ds(0, 16)][0]
        e1 = rp_v[pl.ds(ROWS * 16, 16)][0]
        base0 = (e0 // CA) * CA
        nsuper = (e1 - base0 + ECAP - 1) // ECAP

        def per_edge(masked, gbase, gi, carry):
            r, row_end, a0, a1 = carry
            gidx = gbase + gi
            y0 = ybuf_v[gi, pl.ds(0, 16)]
            y1 = ybuf_v[gi, pl.ds(16, 16)]
            if masked:
                vm = jnp.logical_and(gidx >= e0, gidx < e1)
                vmv = jnp.broadcast_to(
                    jnp.where(vm, jnp.float32(1.0), jnp.float32(0.0)), (16,))
                y0 = y0 * vmv
                y1 = y1 * vmv
            a0 = a0 + y0
            a1 = a1 + y1
            is_end = (gidx + 1) == row_end
            if masked:
                is_end = jnp.logical_and(is_end, vm)

            def flush(a0_, a1_):
                sv = jnp.broadcast_to(dis_v[pl.ds(r * 16, 16)][0], (16,))
                z0 = (0.9 * sv) * a0_ + 0.1 * h_v[pl.ds(r * H2, 16)]
                z1 = (0.9 * sv) * a1_ + 0.1 * h_v[pl.ds(r * H2 + 16, 16)]
                z_v[pl.ds(r * H2, 16)] = z0
                z_v[pl.ds(r * H2 + 16, 16)] = z1
                yo_v[pl.ds(r * H2, 16)] = sv * z0
                yo_v[pl.ds(r * H2 + 16, 16)] = sv * z1
                r2 = r + 1
                row_end2 = rp_v[pl.ds((r2 + 1) * 16, 16)][0]
                return (r2, row_end2, zed, zed)

            def keep(a0_, a1_):
                return (r, row_end, a0_, a1_)

            return lax.cond(is_end, flush, keep, a0, a1)

        def super_body(s, carry):
            sb = base0 + s * ECAP
            pltpu.sync_copy(srcp_hbm.at[pl.ds(sb, ECAP)], srcall_v)
            cnt = jnp.minimum(ECAP, e1 - sb)
            nchs = (cnt + CA - 1) // CA

            def proc_chunk(carry, coff, masked):
                pltpu.async_copy(
                    y_hbm.at[srcall_v.at[pl.ds(coff, CA)]],
                    ybuf_v, sem).wait()
                gbase = sb + coff

                def group(g, cr):
                    for i in range(16):
                        cr = per_edge(masked, gbase, g * 16 + i, cr)
                    return cr

                return lax.fori_loop(0, CA // 16, group, carry)

            carry = proc_chunk(carry, 0, True)
            carry = lax.fori_loop(
                1, nchs - 1,
                lambda c, cr: proc_chunk(cr, c * CA, False), carry)
            carry = lax.fori_loop(
                jnp.maximum(nchs - 1, 1), nchs,
                lambda c, cr: proc_chunk(cr, c * CA, True), carry)
            return carry

        init = (jnp.int32(0), rp_v[pl.ds(16, 16)][0], zed, zed)
        lax.fori_loop(0, nsuper, super_body, init)
        pltpu.sync_copy(z_v, z_hbm.at[pl.ds(d0 * H2, ROWS * H2)])
        pltpu.sync_copy(yo_v, yo_hbm.at[pl.ds(d0 * H2, ROWS * H2)])

    return appnp_sc


_appnp_sc = _make_appnp_sc()


def _pool_final_body(z_ref, aux4_ref, batch_ref, gbc_ref,
                     wgz1_ref, wga1_ref, bg11_ref, wg12_ref, bg12_ref,
                     wnz1_ref, wna1_ref, bn11_ref, wn12_ref, bn12_ref,
                     wgz2_ref, wga2_ref, bg21_ref, wg22_ref, bg22_ref,
                     wnz2_ref, wna2_ref, bn21_ref, wn22_ref, bn22_ref,
                     wgz3_ref, wga3_ref, bg31_ref, wg32_ref, bg32_ref,
                     wnz3_ref, wna3_ref, bn31_ref, wn32_ref, bn32_ref,
                     w3z_ref, w3g_ref, w3a_ref, w3b_ref, b3_ref,
                     w4_ref, b4_ref, w5_ref, b5_ref,
                     out_ref):
    z = z_ref[...]
    aux4 = aux4_ref[...]
    aux3 = aux4[:, 0:3]
    batch = batch_ref[...]
    iota8 = lax.broadcasted_iota(jnp.int32, (1, B), 1)
    mask = (batch == iota8).astype(jnp.float32)

    def pool(wgz, wga, bg1, wg2, bg2, wnz, wna, bn1, wn2, bn2):
        gate = jnp.maximum(z @ wgz + aux3 @ wga + bg1, 0.0) @ wg2 + bg2
        v = jnp.maximum(z @ wnz + aux3 @ wna + bn1, 0.0) @ wn2 + bn2
        gm = jnp.where(mask > 0, gate, -1e30)
        m = jnp.max(gm, axis=0)
        m = jnp.where(m > -1e29, m, 0.0)
        ex = jnp.exp(gate - mask @ m[:, None])
        den = lax.dot_general(mask, ex, (((0,), (0,)), ((), ())))
        a = ex / (mask @ den + 1e-16)
        return lax.dot_general(mask, a * v, (((0,), (0,)), ((), ())))

    g1 = pool(wgz1_ref[...], wga1_ref[...], bg11_ref[...], wg12_ref[...],
              bg12_ref[...], wnz1_ref[...], wna1_ref[...], bn11_ref[...],
              wn12_ref[...], bn12_ref[...])
    g2 = pool(wgz2_ref[...], wga2_ref[...], bg21_ref[...], wg22_ref[...],
              bg22_ref[...], wnz2_ref[...], wna2_ref[...], bn21_ref[...],
              wn22_ref[...], bn22_ref[...])
    g3 = pool(wgz3_ref[...], wga3_ref[...], bg31_ref[...], wg32_ref[...],
              bg32_ref[...], wnz3_ref[...], wna3_ref[...], bn31_ref[...],
              wn32_ref[...], bn32_ref[...])
    g = jnp.concatenate([g1, g2, g3], axis=1)
    tg = g @ w3g_ref[...] + gbc_ref[...] @ w3b_ref[...]
    s = _lrelu(z @ w3z_ref[...] + aux4 @ w3a_ref[...] + mask @ tg
               + b3_ref[...])
    s = _lrelu(s @ w4_ref[...] + b4_ref[...])
    s = s @ w5_ref[...] + b5_ref[...]
    s = jax.nn.sigmoid(s)
    out_ref[...] = lax.dot_general(mask, s, (((0,), (0,)), ((), ())))


def _pool_final(z, aux4, batch, gbc, pool_ws, w3z, w3g, w3a, w3b, b3,
                w4, b4, w5, b5):
    flat = []
    for ws in pool_ws:
        flat.extend(ws)
    return pl.pallas_call(
        _pool_final_body,
        out_shape=jax.ShapeDtypeStruct((B, 1), jnp.float32),
    )(z, aux4, batch, gbc, *flat, w3z, w3g, w3a, w3b, b3, w4, b4, w5, b5)


def _block_diag_att(att_src, att_dst):
    heads, ch = att_src.shape
    eye = jnp.eye(heads, dtype=att_src.dtype)
    a_s = jnp.einsum("hc,hk->hck", att_src, eye).reshape(heads * ch, heads)
    a_d = jnp.einsum("hc,hk->hck", att_dst, eye).reshape(heads * ch, heads)
    return jnp.concatenate([a_s, a_d], axis=1)


def kernel(x, edge_index, batch, J, saved_nodes, infected_nodes,
           size_connected, Omegas, Phis, Lambdas,
           W_gat1, att_src1, att_dst1, b_gat1, W_lin1, b_lin1,
           W_gat2, att_src2, att_dst2, b_gat2, W_lin2, b_lin2,
           Wg11, bg11, Wg12, bg12, Wn11, bn11, Wn12, bn12,
           Wg21, bg21, Wg22, bg22, Wn21, bn21, Wn22, bn22,
           Wg31, bg31, Wg32, bg32, Wn31, bn31, Wn32, bn32,
           W_lin3, b_lin3, W_lin4, b_lin4, W_lin5, b_lin5):
    idt = edge_index.dtype
    loop = jnp.arange(N, dtype=idt)
    src = jnp.concatenate([edge_index[0], loop])
    dst = jnp.concatenate([edge_index[1], loop])
    dst_s, src_s = lax.sort((dst, src), num_keys=1)

    fake_dst = jnp.arange(N, NP_, dtype=idt)
    dst_f = jnp.concatenate([dst_s, fake_dst])
    src_f = jnp.concatenate([src_s, jnp.zeros((NP_ - N,), idt)])
    npad = EP - src_f.shape[0]
    srcp = jnp.concatenate([src_f, jnp.zeros((npad,), idt)])

    rpf = jnp.searchsorted(
        dst_f, jnp.arange(NP_ + 16, dtype=idt)).astype(jnp.int32)
    rp_rep = jnp.repeat(rpf, 16)
    rp = jnp.searchsorted(dst_s, jnp.arange(N + 1, dtype=idt))
    deg = (rp[1:] - rp[:-1]).astype(jnp.float32)[:, None]
    deg = jnp.concatenate([deg, jnp.zeros((NP_ - N, 1), jnp.float32)])

    zrows = jnp.zeros((NP_ - N, 1), jnp.float32)
    x_p = jnp.concatenate([x, jnp.zeros((NP_ - N, IN_DIM), jnp.float32)])
    aux_p = jnp.concatenate([
        jnp.concatenate([J, zrows]), jnp.concatenate([size_connected, zrows]),
        jnp.zeros((NP_, 6), jnp.float32)], axis=1)

    wx = W_gat1[:IN_DIM]
    wjs = jnp.concatenate(
        [W_gat1[IN_DIM:], jnp.zeros((6, HEADS * H1), jnp.float32)])
    asd1 = _block_diag_att(att_src1, att_dst1)
    asd2 = _block_diag_att(att_src2, att_dst2)

    eye4 = jnp.eye(HEADS, dtype=jnp.float32)
    r1 = jnp.repeat(eye4, H1, axis=1)
    r2 = jnp.repeat(eye4, H2, axis=1)

    hrow1, edp1 = _dense1(x_p, aux_p, wx, wjs, asd1)
    out1, den1 = _gat_sc_256(hrow1, edp1.reshape(-1), rp_rep, srcp)
    hrow2, edp2 = _dense2(out1.reshape(NP_, 256), den1.reshape(NP_, 16), r1,
                          b_gat1, W_lin1, b_lin1, W_gat2, asd2)
    out2, den2 = _gat_sc_128(hrow2, edp2.reshape(-1), rp_rep, srcp)
    h32, dis16, y = _dense3(out2.reshape(NP_, 128), den2.reshape(NP_, 16), r2,
                            deg, b_gat2, W_lin2, b_lin2)

    h32f = h32.reshape(-1)
    dis16f = dis16.reshape(-1)
    z = None
    for _ in range(K_POWER):
        zf, yf = _appnp_sc(y, h32f, dis16f, rp_rep, srcp)
        y = yf.reshape(NP_, H2)
        z = zf.reshape(NP_, H2)

    aux4 = jnp.concatenate([J, saved_nodes, infected_nodes, size_connected],
                           axis=1)
    gbc = jnp.concatenate([Omegas, Phis, Lambdas], axis=1)
    pool_ws = [
        (Wg11[:H2], Wg11[H2:], bg11, Wg12, bg12,
         Wn11[:H2], Wn11[H2:], bn11, Wn12, bn12),
        (Wg21[:H2], Wg21[H2:], bg21, Wg22, bg22,
         Wn21[:H2], Wn21[H2:], bn21, Wn22, bn22),
        (Wg31[:H2], Wg31[H2:], bg31, Wg32, bg32,
         Wn31[:H2], Wn31[H2:], bn31, Wn32, bn32),
    ]
    w3z = W_lin3[0:H2]
    w3g = W_lin3[H2:H2 + 3 * H2]
    w3a = W_lin3[H2 + 3 * H2:H2 + 3 * H2 + 4]
    w3b = W_lin3[H2 + 3 * H2 + 4:]
    return _pool_final(z[:N], aux4, batch[:, None], gbc, pool_ws,
                       w3z, w3g, w3a, w3b, b_lin3,
                       W_lin4, b_lin4, W_lin5, b_lin5)


# speedup vs baseline: 18.4302x; 1.1925x over previous
import functools

import jax
import jax.numpy as jnp
from jax import lax
from jax.experimental import pallas as pl
from jax.experimental.pallas import tpu as pltpu
from jax.experimental.pallas import tpu_sc as plsc

N = 10000
E = 160000
B = 8
IN_DIM = 128
H1 = 64
H2 = 32
HEADS = 4
K_POWER = 10
ALPHA = 0.1

NP_ = 10240
TILES = 32
ROWS = NP_ // TILES
ECAP = 6144
EP = 170240 + ECAP
CG = 64
CA = 128
BLK = 1024


def _lrelu(t):
    return jnp.where(t >= 0, t, 0.2 * t)


def _dense1_body(x_ref, aux_ref, wx_ref, wjs_ref, asd_ref, hrow_ref, ed_ref):
    h = x_ref[...] @ wx_ref[...] + aux_ref[...] @ wjs_ref[...]
    esed = h @ asd_ref[...]
    zpad = jnp.zeros((h.shape[0], 12), jnp.float32)
    hrow_ref[...] = jnp.concatenate([h, esed[:, 0:4], zpad], axis=1)
    ed_ref[...] = jnp.concatenate([esed[:, 4:8], zpad], axis=1)


def _dense1(x_p, aux_p, wx, wjs, asd):
    g = NP_ // BLK
    return pl.pallas_call(
        _dense1_body,
        grid=(g,),
        in_specs=[
            pl.BlockSpec((BLK, IN_DIM), lambda i: (i, 0)),
            pl.BlockSpec((BLK, 8), lambda i: (i, 0)),
            pl.BlockSpec(wx.shape, lambda i: (0, 0)),
            pl.BlockSpec(wjs.shape, lambda i: (0, 0)),
            pl.BlockSpec(asd.shape, lambda i: (0, 0)),
        ],
        out_specs=[
            pl.BlockSpec((BLK, 272), lambda i: (i, 0)),
            pl.BlockSpec((BLK, 16), lambda i: (i, 0)),
        ],
        out_shape=[
            jax.ShapeDtypeStruct((NP_, 272), jnp.float32),
            jax.ShapeDtypeStruct((NP_, 16), jnp.float32),
        ],
    )(x_p, aux_p, wx, wjs, asd)


def _dense2_body(o_ref, den_ref, r1_ref, bg_ref, wl_ref, bl_ref, wg_ref,
                 asd_ref, hrow_ref, ed_ref):
    inv4 = 1.0 / (den_ref[...][:, 0:4] + 1e-16)
    o = o_ref[...] * (inv4 @ r1_ref[...])
    t = _lrelu((o + bg_ref[...]) @ wl_ref[...] + bl_ref[...])
    h2 = t @ wg_ref[...]
    esed = h2 @ asd_ref[...]
    zpad = jnp.zeros((h2.shape[0], 12), jnp.float32)
    hrow_ref[...] = jnp.concatenate([h2, esed[:, 0:4], zpad], axis=1)
    ed_ref[...] = jnp.concatenate([esed[:, 4:8], zpad], axis=1)


def _dense2(out1, den1, r1, b_gat1, W_lin1, b_lin1, W_gat2, asd2):
    g = NP_ // BLK
    return pl.pallas_call(
        _dense2_body,
        grid=(g,),
        in_specs=[
            pl.BlockSpec((BLK, 256), lambda i: (i, 0)),
            pl.BlockSpec((BLK, 16), lambda i: (i, 0)),
            pl.BlockSpec((HEADS, 256), lambda i: (0, 0)),
            pl.BlockSpec((1, 256), lambda i: (0, 0)),
            pl.BlockSpec(W_lin1.shape, lambda i: (0, 0)),
            pl.BlockSpec((1, H1), lambda i: (0, 0)),
            pl.BlockSpec(W_gat2.shape, lambda i: (0, 0)),
            pl.BlockSpec(asd2.shape, lambda i: (0, 0)),
        ],
        out_specs=[
            pl.BlockSpec((BLK, 144), lambda i: (i, 0)),
            pl.BlockSpec((BLK, 16), lambda i: (i, 0)),
        ],
        out_shape=[
            jax.ShapeDtypeStruct((NP_, 144), jnp.float32),
            jax.ShapeDtypeStruct((NP_, 16), jnp.float32),
        ],
    )(out1, den1, r1, b_gat1[None, :], W_lin1, b_lin1[None, :], W_gat2, asd2)


def _dense3_body(o_ref, den_ref, r2_ref, deg_ref, bg_ref, wl_ref, bl_ref,
                 h_ref, dis_ref, y_ref):
    inv4 = 1.0 / (den_ref[...][:, 0:4] + 1e-16)
    o = o_ref[...] * (inv4 @ r2_ref[...])
    h32 = _lrelu((o + bg_ref[...]) @ wl_ref[...] + bl_ref[...])
    deg = deg_ref[...]
    dis = jnp.where(deg > 0, lax.rsqrt(jnp.maximum(deg, 1e-12)), 0.0)
    h_ref[...] = h32
    dis_ref[...] = jnp.concatenate(
        [dis, jnp.zeros((dis.shape[0], 15), jnp.float32)], axis=1)
    y_ref[...] = h32 * dis


def _dense3(out2, den2, r2, deg, b_gat2, W_lin2, b_lin2):
    g = NP_ // BLK
    return pl.pallas_call(
        _dense3_body,
        grid=(g,),
        in_specs=[
            pl.BlockSpec((BLK, 128), lambda i: (i, 0)),
            pl.BlockSpec((BLK, 16), lambda i: (i, 0)),
            pl.BlockSpec((HEADS, 128), lambda i: (0, 0)),
            pl.BlockSpec((BLK, 1), lambda i: (i, 0)),
            pl.BlockSpec((1, 128), lambda i: (0, 0)),
            pl.BlockSpec(W_lin2.shape, lambda i: (0, 0)),
            pl.BlockSpec((1, H2), lambda i: (0, 0)),
        ],
        out_specs=[
            pl.BlockSpec((BLK, H2), lambda i: (i, 0)),
            pl.BlockSpec((BLK, 16), lambda i: (i, 0)),
            pl.BlockSpec((BLK, H2), lambda i: (i, 0)),
        ],
        out_shape=[
            jax.ShapeDtypeStruct((NP_, H2), jnp.float32),
            jax.ShapeDtypeStruct((NP_, 16), jnp.float32),
            jax.ShapeDtypeStruct((NP_, H2), jnp.float32),
        ],
    )(out2, den2, r2, deg, b_gat2[None, :], W_lin2, b_lin2[None, :])


def _bcast_lane(v, i):
    return jnp.broadcast_to(v[i], (16,))


def _make_gat_sc(ch):
    roww = ch + 16
    cph = ch // HEADS
    sub = cph // 16
    nacc = HEADS * sub
    mesh = plsc.VectorSubcoreMesh(core_axis_name="c", subcore_axis_name="s")

    @functools.partial(
        pl.kernel,
        mesh=mesh,
        out_type=[
            jax.ShapeDtypeStruct((NP_ * ch,), jnp.float32),
            jax.ShapeDtypeStruct((NP_ * 16,), jnp.float32),
        ],
        compiler_params=pltpu.CompilerParams(use_tc_tiling_on_sc=False),
        scratch_types=[
            pltpu.VMEM(((ROWS + 16) * 16,), jnp.int32),
            pltpu.VMEM((ROWS * 16,), jnp.float32),
            pltpu.VMEM((ECAP,), jnp.int32),
            pltpu.VMEM((ECAP,), jnp.int32),
            pltpu.VMEM((CG, roww), jnp.float32),
            pltpu.VMEM((ROWS * ch,), jnp.float32),
            pltpu.VMEM((ROWS * 16,), jnp.float32),
            pltpu.SemaphoreType.DMA,
        ],
    )
    def gat_sc(hrow_hbm, edp_hbm, rp_hbm, srcp_hbm, dstp_hbm,
               out_hbm, deno_hbm,
               rp_v, ed_v, srcall_v, dstall_v, gbuf_v, out_v, den_v, sem):
        zed = jnp.zeros((16,), jnp.float32)
        wid = lax.axis_index("s") * 2 + lax.axis_index("c")
        d0 = wid * ROWS
        pltpu.sync_copy(rp_hbm.at[pl.ds(d0 * 16, (ROWS + 16) * 16)], rp_v)
        pltpu.sync_copy(edp_hbm.at[pl.ds(d0 * 16, ROWS * 16)], ed_v)
        e0 = rp_v[pl.ds(0, 16)][0]
        e1 = rp_v[pl.ds(ROWS * 16, 16)][0]
        base0 = (e0 // CG) * CG
        nsuper = (e1 - base0 + ECAP - 1) // ECAP

        def zero_row(r, _):
            den_v[pl.ds(r * 16, 16)] = zed
            for k in range(nacc):
                out_v[pl.ds(r * ch + k * 16, 16)] = zed
            return 0

        lax.fori_loop(0, ROWS, zero_row, 0)

        def per_edge(masked, gbase, gi, dvec, lane):
            dl = dvec[lane] - d0
            esv = gbuf_v[gi, pl.ds(ch, 16)]
            if masked:
                gidx = gbase + gi
                vm = jnp.logical_and(gidx >= e0, gidx < e1)
                dl = jnp.where(vm, dl, 0)
            edrow = ed_v[pl.ds(dl * 16, 16)]
            ex = jnp.exp(_lrelu(esv + edrow))
            if masked:
                ex = jnp.where(vm, ex, zed)
            plsc.addupdate(den_v.at[pl.ds(dl * 16, 16)], ex)
            for h in range(HEADS):
                w = _bcast_lane(ex, h)
                for j in range(sub):
                    plsc.addupdate(
                        out_v.at[pl.ds(dl * ch + h * cph + j * 16, 16)],
                        w * gbuf_v[gi, pl.ds(h * cph + j * 16, 16)])

        def super_body(s, _):
            sb = base0 + s * ECAP
            pltpu.sync_copy(srcp_hbm.at[pl.ds(sb, ECAP)], srcall_v)
            pltpu.sync_copy(dstp_hbm.at[pl.ds(sb, ECAP)], dstall_v)
            cnt = jnp.minimum(ECAP, e1 - sb)
            nchs = (cnt + CG - 1) // CG

            def proc_chunk(coff, masked):
                pltpu.async_copy(
                    hrow_hbm.at[srcall_v.at[pl.ds(coff, CG)]],
                    gbuf_v, sem).wait()
                gbase = sb + coff

                def group(g, _):
                    dvec = dstall_v[pl.ds(coff + g * 16, 16)]
                    for i in range(16):
                        per_edge(masked, gbase, g * 16 + i, dvec, i)
                    return 0

                lax.fori_loop(0, CG // 16, group, 0)

            proc_chunk(0, True)
            lax.fori_loop(
                1, nchs - 1,
                lambda c, cr: (proc_chunk(c * CG, False), 0)[1], 0)
            lax.fori_loop(
                jnp.maximum(nchs - 1, 1), nchs,
                lambda c, cr: (proc_chunk(c * CG, True), 0)[1], 0)
            return 0

        lax.fori_loop(0, nsuper, super_body, 0)
        pltpu.sync_copy(out_v, out_hbm.at[pl.ds(d0 * ch, ROWS * ch)])
        pltpu.sync_copy(den_v, deno_hbm.at[pl.ds(d0 * 16, ROWS * 16)])

    return gat_sc


_gat_sc_256 = _make_gat_sc(256)
_gat_sc_128 = _make_gat_sc(128)


def _make_appnp_sc():
    mesh = plsc.VectorSubcoreMesh(core_axis_name="c", subcore_axis_name="s")

    @functools.partial(
        pl.kernel,
        mesh=mesh,
        out_type=[
            jax.ShapeDtypeStruct((NP_ * H2,), jnp.float32),
            jax.ShapeDtypeStruct((NP_ * H2,), jnp.float32),
        ],
        compiler_params=pltpu.CompilerParams(use_tc_tiling_on_sc=False),
        scratch_types=[
            pltpu.VMEM(((ROWS + 16) * 16,), jnp.int32),
            pltpu.VMEM((ROWS * H2,), jnp.float32),
            pltpu.VMEM((ROWS * 16,), jnp.float32),
            pltpu.VMEM((ECAP,), jnp.int32),
            pltpu.VMEM((ECAP,), jnp.int32),
            pltpu.VMEM((CA, H2), jnp.float32),
            pltpu.VMEM((ROWS * H2,), jnp.float32),
            pltpu.VMEM((ROWS * H2,), jnp.float32),
            pltpu.VMEM((ROWS * H2,), jnp.float32),
            pltpu.SemaphoreType.DMA,
        ],
    )
    def appnp_sc(y_hbm, h_hbm, dis_hbm, rp_hbm, srcp_hbm, dstp_hbm,
                 z_hbm, yo_hbm,
                 rp_v, h_v, dis_v, srcall_v, dstall_v, ybuf_v,
                 acc_v, z_v, yo_v, sem):
        zed = jnp.zeros((16,), jnp.float32)
        wid = lax.axis_index("s") * 2 + lax.axis_index("c")
        d0 = wid * ROWS
        pltpu.sync_copy(rp_hbm.at[pl.ds(d0 * 16, (ROWS + 16) * 16)], rp_v)
        pltpu.sync_copy(h_hbm.at[pl.ds(d0 * H2, ROWS * H2)], h_v)
        pltpu.sync_copy(dis_hbm.at[pl.ds(d0 * 16, ROWS * 16)], dis_v)
        e0 = rp_v[pl.ds(0, 16)][0]
        e1 = rp_v[pl.ds(ROWS * 16, 16)][0]
        base0 = (e0 // CA) * CA
        nsuper = (e1 - base0 + ECAP - 1) // ECAP

        def zero_row(r, _):
            acc_v[pl.ds(r * H2, 16)] = zed
            acc_v[pl.ds(r * H2 + 16, 16)] = zed
            return 0

        lax.fori_loop(0, ROWS, zero_row, 0)

        def per_edge(masked, gbase, gi, dvec, lane):
            dl = dvec[lane] - d0
            y0 = ybuf_v[gi, pl.ds(0, 16)]
            y1 = ybuf_v[gi, pl.ds(16, 16)]
            if masked:
                gidx = gbase + gi
                vm = jnp.logical_and(gidx >= e0, gidx < e1)
                vmv = jnp.broadcast_to(
                    jnp.where(vm, jnp.float32(1.0), jnp.float32(0.0)), (16,))
                y0 = y0 * vmv
                y1 = y1 * vmv
                dl = jnp.where(vm, dl, 0)
            plsc.addupdate(acc_v.at[pl.ds(dl * H2, 16)], y0)
            plsc.addupdate(acc_v.at[pl.ds(dl * H2 + 16, 16)], y1)

        def super_body(s, _):
            sb = base0 + s * ECAP
            pltpu.sync_copy(srcp_hbm.at[pl.ds(sb, ECAP)], srcall_v)
            pltpu.sync_copy(dstp_hbm.at[pl.ds(sb, ECAP)], dstall_v)
            cnt = jnp.minimum(ECAP, e1 - sb)
            nchs = (cnt + CA - 1) // CA

            def proc_chunk(coff, masked):
                pltpu.async_copy(
                    y_hbm.at[srcall_v.at[pl.ds(coff, CA)]],
                    ybuf_v, sem).wait()
                gbase = sb + coff

                def group(g, _):
                    dvec = dstall_v[pl.ds(coff + g * 16, 16)]
                    for i in range(16):
                        per_edge(masked, gbase, g * 16 + i, dvec, i)
                    return 0

                lax.fori_loop(0, CA // 16, group, 0)

            proc_chunk(0, True)
            lax.fori_loop(
                1, nchs - 1,
                lambda c, cr: (proc_chunk(c * CA, False), 0)[1], 0)
            lax.fori_loop(
                jnp.maximum(nchs - 1, 1), nchs,
                lambda c, cr: (proc_chunk(c * CA, True), 0)[1], 0)
            return 0

        lax.fori_loop(0, nsuper, super_body, 0)

        def post_row(r, _):
            sv = jnp.broadcast_to(dis_v[pl.ds(r * 16, 16)][0], (16,))
            a0 = acc_v[pl.ds(r * H2, 16)]
            a1 = acc_v[pl.ds(r * H2 + 16, 16)]
            z0 = (0.9 * sv) * a0 + 0.1 * h_v[pl.ds(r * H2, 16)]
            z1 = (0.9 * sv) * a1 + 0.1 * h_v[pl.ds(r * H2 + 16, 16)]
            z_v[pl.ds(r * H2, 16)] = z0
            z_v[pl.ds(r * H2 + 16, 16)] = z1
            yo_v[pl.ds(r * H2, 16)] = sv * z0
            yo_v[pl.ds(r * H2 + 16, 16)] = sv * z1
            return 0

        lax.fori_loop(0, ROWS, post_row, 0)
        pltpu.sync_copy(z_v, z_hbm.at[pl.ds(d0 * H2, ROWS * H2)])
        pltpu.sync_copy(yo_v, yo_hbm.at[pl.ds(d0 * H2, ROWS * H2)])

    return appnp_sc


_appnp_sc = _make_appnp_sc()


def _pool_final_body(z_ref, aux4_ref, batch_ref, gbc_ref,
                     wgz1_ref, wga1_ref, bg11_ref, wg12_ref, bg12_ref,
                     wnz1_ref, wna1_ref, bn11_ref, wn12_ref, bn12_ref,
                     wgz2_ref, wga2_ref, bg21_ref, wg22_ref, bg22_ref,
                     wnz2_ref, wna2_ref, bn21_ref, wn22_ref, bn22_ref,
                     wgz3_ref, wga3_ref, bg31_ref, wg32_ref, bg32_ref,
                     wnz3_ref, wna3_ref, bn31_ref, wn32_ref, bn32_ref,
                     w3z_ref, w3g_ref, w3a_ref, w3b_ref, b3_ref,
                     w4_ref, b4_ref, w5_ref, b5_ref,
                     out_ref):
    z = z_ref[...]
    aux4 = aux4_ref[...]
    aux3 = aux4[:, 0:3]
    batch = batch_ref[...]
    iota8 = lax.broadcasted_iota(jnp.int32, (1, B), 1)
    mask = (batch == iota8).astype(jnp.float32)

    def pool(wgz, wga, bg1, wg2, bg2, wnz, wna, bn1, wn2, bn2):
        gate = jnp.maximum(z @ wgz + aux3 @ wga + bg1, 0.0) @ wg2 + bg2
        v = jnp.maximum(z @ wnz + aux3 @ wna + bn1, 0.0) @ wn2 + bn2
        gm = jnp.where(mask > 0, gate, -1e30)
        m = jnp.max(gm, axis=0)
        m = jnp.where(m > -1e29, m, 0.0)
        ex = jnp.exp(gate - mask @ m[:, None])
        den = lax.dot_general(mask, ex, (((0,), (0,)), ((), ())))
        a = ex / (mask @ den + 1e-16)
        return lax.dot_general(mask, a * v, (((0,), (0,)), ((), ())))

    g1 = pool(wgz1_ref[...], wga1_ref[...], bg11_ref[...], wg12_ref[...],
              bg12_ref[...], wnz1_ref[...], wna1_ref[...], bn11_ref[...],
              wn12_ref[...], bn12_ref[...])
    g2 = pool(wgz2_ref[...], wga2_ref[...], bg21_ref[...], wg22_ref[...],
              bg22_ref[...], wnz2_ref[...], wna2_ref[...], bn21_ref[...],
              wn22_ref[...], bn22_ref[...])
    g3 = pool(wgz3_ref[...], wga3_ref[...], bg31_ref[...], wg32_ref[...],
              bg32_ref[...], wnz3_ref[...], wna3_ref[...], bn31_ref[...],
              wn32_ref[...], bn32_ref[...])
    g = jnp.concatenate([g1, g2, g3], axis=1)
    tg = g @ w3g_ref[...] + gbc_ref[...] @ w3b_ref[...]
    s = _lrelu(z @ w3z_ref[...] + aux4 @ w3a_ref[...] + mask @ tg
               + b3_ref[...])
    s = _lrelu(s @ w4_ref[...] + b4_ref[...])
    s = s @ w5_ref[...] + b5_ref[...]
    s = jax.nn.sigmoid(s)
    out_ref[...] = lax.dot_general(mask, s, (((0,), (0,)), ((), ())))


def _pool_final(z, aux4, batch, gbc, pool_ws, w3z, w3g, w3a, w3b, b3,
                w4, b4, w5, b5):
    flat = []
    for ws in pool_ws:
        flat.extend(ws)
    return pl.pallas_call(
        _pool_final_body,
        out_shape=jax.ShapeDtypeStruct((B, 1), jnp.float32),
    )(z, aux4, batch, gbc, *flat, w3z, w3g, w3a, w3b, b3, w4, b4, w5, b5)


def _block_diag_att(att_src, att_dst):
    heads, ch = att_src.shape
    eye = jnp.eye(heads, dtype=att_src.dtype)
    a_s = jnp.einsum("hc,hk->hck", att_src, eye).reshape(heads * ch, heads)
    a_d = jnp.einsum("hc,hk->hck", att_dst, eye).reshape(heads * ch, heads)
    return jnp.concatenate([a_s, a_d], axis=1)


def kernel(x, edge_index, batch, J, saved_nodes, infected_nodes,
           size_connected, Omegas, Phis, Lambdas,
           W_gat1, att_src1, att_dst1, b_gat1, W_lin1, b_lin1,
           W_gat2, att_src2, att_dst2, b_gat2, W_lin2, b_lin2,
           Wg11, bg11, Wg12, bg12, Wn11, bn11, Wn12, bn12,
           Wg21, bg21, Wg22, bg22, Wn21, bn21, Wn22, bn22,
           Wg31, bg31, Wg32, bg32, Wn31, bn31, Wn32, bn32,
           W_lin3, b_lin3, W_lin4, b_lin4, W_lin5, b_lin5):
    idt = edge_index.dtype
    loop = jnp.arange(N, dtype=idt)
    src = jnp.concatenate([edge_index[0], loop])
    dst = jnp.concatenate([edge_index[1], loop])
    dst_s, src_s = lax.sort((dst, src), num_keys=1)

    fake_dst = jnp.arange(N, NP_, dtype=idt)
    dst_f = jnp.concatenate([dst_s, fake_dst])
    src_f = jnp.concatenate([src_s, jnp.zeros((NP_ - N,), idt)])
    npad = EP - src_f.shape[0]
    srcp = jnp.concatenate([src_f, jnp.zeros((npad,), idt)])
    dstp = jnp.concatenate([dst_f, jnp.zeros((npad,), idt)])

    rpf = jnp.searchsorted(
        dst_f, jnp.arange(NP_ + 16, dtype=idt)).astype(jnp.int32)
    rp_rep = jnp.repeat(rpf, 16)
    rp = jnp.searchsorted(dst_s, jnp.arange(N + 1, dtype=idt))
    deg = (rp[1:] - rp[:-1]).astype(jnp.float32)[:, None]
    deg = jnp.concatenate([deg, jnp.zeros((NP_ - N, 1), jnp.float32)])

    zrows = jnp.zeros((NP_ - N, 1), jnp.float32)
    x_p = jnp.concatenate([x, jnp.zeros((NP_ - N, IN_DIM), jnp.float32)])
    aux_p = jnp.concatenate([
        jnp.concatenate([J, zrows]), jnp.concatenate([size_connected, zrows]),
        jnp.zeros((NP_, 6), jnp.float32)], axis=1)

    wx = W_gat1[:IN_DIM]
    wjs = jnp.concatenate(
        [W_gat1[IN_DIM:], jnp.zeros((6, HEADS * H1), jnp.float32)])
    asd1 = _block_diag_att(att_src1, att_dst1)
    asd2 = _block_diag_att(att_src2, att_dst2)

    eye4 = jnp.eye(HEADS, dtype=jnp.float32)
    r1 = jnp.repeat(eye4, H1, axis=1)
    r2 = jnp.repeat(eye4, H2, axis=1)

    hrow1, edp1 = _dense1(x_p, aux_p, wx, wjs, asd1)
    out1, den1 = _gat_sc_256(hrow1, edp1.reshape(-1), rp_rep, srcp, dstp)
    hrow2, edp2 = _dense2(out1.reshape(NP_, 256), den1.reshape(NP_, 16), r1,
                          b_gat1, W_lin1, b_lin1, W_gat2, asd2)
    out2, den2 = _gat_sc_128(hrow2, edp2.reshape(-1), rp_rep, srcp, dstp)
    h32, dis16, y = _dense3(out2.reshape(NP_, 128), den2.reshape(NP_, 16), r2,
                            deg, b_gat2, W_lin2, b_lin2)

    h32f = h32.reshape(-1)
    dis16f = dis16.reshape(-1)
    z = None
    for _ in range(K_POWER):
        zf, yf = _appnp_sc(y, h32f, dis16f, rp_rep, srcp, dstp)
        y = yf.reshape(NP_, H2)
        z = zf.reshape(NP_, H2)

    aux4 = jnp.concatenate([J, saved_nodes, infected_nodes, size_connected],
                           axis=1)
    gbc = jnp.concatenate([Omegas, Phis, Lambdas], axis=1)
    pool_ws = [
        (Wg11[:H2], Wg11[H2:], bg11, Wg12, bg12,
         Wn11[:H2], Wn11[H2:], bn11, Wn12, bn12),
        (Wg21[:H2], Wg21[H2:], bg21, Wg22, bg22,
         Wn21[:H2], Wn21[H2:], bn21, Wn22, bn22),
        (Wg31[:H2], Wg31[H2:], bg31, Wg32, bg32,
         Wn31[:H2], Wn31[H2:], bn31, Wn32, bn32),
    ]
    w3z = W_lin3[0:H2]
    w3g = W_lin3[H2:H2 + 3 * H2]
    w3a = W_lin3[H2 + 3 * H2:H2 + 3 * H2 + 4]
    w3b = W_lin3[H2 + 3 * H2 + 4:]
    return _pool_final(z[:N], aux4, batch[:, None], gbc, pool_ws,
                       w3z, w3g, w3a, w3b, b_lin3,
                       W_lin4, b_lin4, W_lin5, b_lin5)


# speedup vs baseline: 20.5289x; 1.1139x over previous
import functools

import jax
import jax.numpy as jnp
from jax import lax
from jax.experimental import pallas as pl
from jax.experimental.pallas import tpu as pltpu
from jax.experimental.pallas import tpu_sc as plsc

N = 10000
E = 160000
B = 8
IN_DIM = 128
H1 = 64
H2 = 32
HEADS = 4
K_POWER = 10
ALPHA = 0.1

NP_ = 10240
TILES = 32
ROWS = NP_ // TILES
ECAP = 6144
EP = 170240 + ECAP
CG = 64
CA = 128
BLK = 1024


def _lrelu(t):
    return jnp.where(t >= 0, t, 0.2 * t)


def _dense1_body(x_ref, aux_ref, wx_ref, wjs_ref, asd_ref, hrow_ref, ed_ref):
    h = x_ref[...] @ wx_ref[...] + aux_ref[...] @ wjs_ref[...]
    esed = h @ asd_ref[...]
    zpad = jnp.zeros((h.shape[0], 12), jnp.float32)
    hrow_ref[...] = jnp.concatenate([h, esed[:, 0:4], zpad], axis=1)
    ed_ref[...] = jnp.concatenate([esed[:, 4:8], zpad], axis=1)


def _dense1(x_p, aux_p, wx, wjs, asd):
    g = NP_ // BLK
    return pl.pallas_call(
        _dense1_body,
        grid=(g,),
        in_specs=[
            pl.BlockSpec((BLK, IN_DIM), lambda i: (i, 0)),
            pl.BlockSpec((BLK, 8), lambda i: (i, 0)),
            pl.BlockSpec(wx.shape, lambda i: (0, 0)),
            pl.BlockSpec(wjs.shape, lambda i: (0, 0)),
            pl.BlockSpec(asd.shape, lambda i: (0, 0)),
        ],
        out_specs=[
            pl.BlockSpec((BLK, 272), lambda i: (i, 0)),
            pl.BlockSpec((BLK, 16), lambda i: (i, 0)),
        ],
        out_shape=[
            jax.ShapeDtypeStruct((NP_, 272), jnp.float32),
            jax.ShapeDtypeStruct((NP_, 16), jnp.float32),
        ],
    )(x_p, aux_p, wx, wjs, asd)


def _dense2_body(o_ref, den_ref, r1_ref, bg_ref, wl_ref, bl_ref, wg_ref,
                 asd_ref, hrow_ref, ed_ref):
    inv4 = 1.0 / (den_ref[...][:, 0:4] + 1e-16)
    o = o_ref[...] * (inv4 @ r1_ref[...])
    t = _lrelu((o + bg_ref[...]) @ wl_ref[...] + bl_ref[...])
    h2 = t @ wg_ref[...]
    esed = h2 @ asd_ref[...]
    zpad = jnp.zeros((h2.shape[0], 12), jnp.float32)
    hrow_ref[...] = jnp.concatenate([h2, esed[:, 0:4], zpad], axis=1)
    ed_ref[...] = jnp.concatenate([esed[:, 4:8], zpad], axis=1)


def _dense2(out1, den1, r1, b_gat1, W_lin1, b_lin1, W_gat2, asd2):
    g = NP_ // BLK
    return pl.pallas_call(
        _dense2_body,
        grid=(g,),
        in_specs=[
            pl.BlockSpec((BLK, 256), lambda i: (i, 0)),
            pl.BlockSpec((BLK, 16), lambda i: (i, 0)),
            pl.BlockSpec((HEADS, 256), lambda i: (0, 0)),
            pl.BlockSpec((1, 256), lambda i: (0, 0)),
            pl.BlockSpec(W_lin1.shape, lambda i: (0, 0)),
            pl.BlockSpec((1, H1), lambda i: (0, 0)),
            pl.BlockSpec(W_gat2.shape, lambda i: (0, 0)),
            pl.BlockSpec(asd2.shape, lambda i: (0, 0)),
        ],
        out_specs=[
            pl.BlockSpec((BLK, 144), lambda i: (i, 0)),
            pl.BlockSpec((BLK, 16), lambda i: (i, 0)),
        ],
        out_shape=[
            jax.ShapeDtypeStruct((NP_, 144), jnp.float32),
            jax.ShapeDtypeStruct((NP_, 16), jnp.float32),
        ],
    )(out1, den1, r1, b_gat1[None, :], W_lin1, b_lin1[None, :], W_gat2, asd2)


def _dense3_body(o_ref, den_ref, r2_ref, deg_ref, bg_ref, wl_ref, bl_ref,
                 h_ref, dis_ref, y_ref):
    inv4 = 1.0 / (den_ref[...][:, 0:4] + 1e-16)
    o = o_ref[...] * (inv4 @ r2_ref[...])
    h32 = _lrelu((o + bg_ref[...]) @ wl_ref[...] + bl_ref[...])
    deg = deg_ref[...]
    dis = jnp.where(deg > 0, lax.rsqrt(jnp.maximum(deg, 1e-12)), 0.0)
    h_ref[...] = h32
    dis_ref[...] = jnp.concatenate(
        [dis, jnp.zeros((dis.shape[0], 15), jnp.float32)], axis=1)
    y_ref[...] = h32 * dis


def _dense3(out2, den2, r2, deg, b_gat2, W_lin2, b_lin2):
    g = NP_ // BLK
    return pl.pallas_call(
        _dense3_body,
        grid=(g,),
        in_specs=[
            pl.BlockSpec((BLK, 128), lambda i: (i, 0)),
            pl.BlockSpec((BLK, 16), lambda i: (i, 0)),
            pl.BlockSpec((HEADS, 128), lambda i: (0, 0)),
            pl.BlockSpec((BLK, 1), lambda i: (i, 0)),
            pl.BlockSpec((1, 128), lambda i: (0, 0)),
            pl.BlockSpec(W_lin2.shape, lambda i: (0, 0)),
            pl.BlockSpec((1, H2), lambda i: (0, 0)),
        ],
        out_specs=[
            pl.BlockSpec((BLK, H2), lambda i: (i, 0)),
            pl.BlockSpec((BLK, 16), lambda i: (i, 0)),
            pl.BlockSpec((BLK, H2), lambda i: (i, 0)),
        ],
        out_shape=[
            jax.ShapeDtypeStruct((NP_, H2), jnp.float32),
            jax.ShapeDtypeStruct((NP_, 16), jnp.float32),
            jax.ShapeDtypeStruct((NP_, H2), jnp.float32),
        ],
    )(out2, den2, r2, deg, b_gat2[None, :], W_lin2, b_lin2[None, :])


def _bcast_lane(v, i):
    return jnp.broadcast_to(v[i], (16,))


def _make_gat_sc(ch):
    roww = ch + 16
    cph = ch // HEADS
    sub = cph // 16
    nacc = HEADS * sub
    mesh = plsc.VectorSubcoreMesh(core_axis_name="c", subcore_axis_name="s")

    @functools.partial(
        pl.kernel,
        mesh=mesh,
        out_type=[
            jax.ShapeDtypeStruct((NP_ * ch,), jnp.float32),
            jax.ShapeDtypeStruct((NP_ * 16,), jnp.float32),
        ],
        compiler_params=pltpu.CompilerParams(use_tc_tiling_on_sc=False),
        scratch_types=[
            pltpu.VMEM(((ROWS + 16) * 16,), jnp.int32),
            pltpu.VMEM((ROWS * 16,), jnp.float32),
            pltpu.VMEM((ECAP,), jnp.int32),
            pltpu.VMEM((CG, roww), jnp.float32),
            pltpu.VMEM((ROWS * ch,), jnp.float32),
            pltpu.VMEM((ROWS * 16,), jnp.float32),
            pltpu.SemaphoreType.DMA,
        ],
    )
    def gat_sc(hrow_hbm, edp_hbm, rp_hbm, srcp_hbm, dstp_hbm,
               out_hbm, deno_hbm,
               rp_v, ed_v, srcall_v, gbuf_v, out_v, den_v, sem):
        zed = jnp.zeros((16,), jnp.float32)
        wid = lax.axis_index("s") * 2 + lax.axis_index("c")
        d0 = wid * ROWS
        pltpu.sync_copy(rp_hbm.at[pl.ds(d0 * 16, (ROWS + 16) * 16)], rp_v)
        pltpu.sync_copy(edp_hbm.at[pl.ds(d0 * 16, ROWS * 16)], ed_v)
        e0 = rp_v[pl.ds(0, 16)][0]
        e1 = rp_v[pl.ds(ROWS * 16, 16)][0]
        base0 = (e0 // CG) * CG
        nsuper = (e1 - base0 + ECAP - 1) // ECAP

        def per_edge(masked, gbase, gi, carry):
            r, row_end, den, accs = carry
            gidx = gbase + gi
            esv = gbuf_v[gi, pl.ds(ch, 16)]
            edrow = ed_v[pl.ds(r * 16, 16)]
            ex = jnp.exp(_lrelu(esv + edrow))
            if masked:
                vm = jnp.logical_and(gidx >= e0, gidx < e1)
                ex = jnp.where(vm, ex, zed)
            den = den + ex
            naccs = []
            for h in range(HEADS):
                w = _bcast_lane(ex, h)
                for j in range(sub):
                    naccs.append(accs[h * sub + j]
                                 + w * gbuf_v[gi, pl.ds(h * cph + j * 16, 16)])
            is_end = (gidx + 1) == row_end
            if masked:
                is_end = jnp.logical_and(is_end, vm)

            def flush(den_, accs_):
                den_v[pl.ds(r * 16, 16)] = den_
                for k in range(nacc):
                    out_v[pl.ds(r * ch + k * 16, 16)] = accs_[k]
                r2 = r + 1
                row_end2 = rp_v[pl.ds((r2 + 1) * 16, 16)][0]
                return (r2, row_end2, zed,
                        tuple(zed for _ in range(nacc)))

            def keep(den_, accs_):
                return (r, row_end, den_, accs_)

            return lax.cond(is_end, flush, keep, den, tuple(naccs))

        def super_body(s, carry):
            sb = base0 + s * ECAP
            pltpu.sync_copy(srcp_hbm.at[pl.ds(sb, ECAP)], srcall_v)
            cnt = jnp.minimum(ECAP, e1 - sb)
            nchs = (cnt + CG - 1) // CG

            def proc_chunk(carry, coff, masked):
                pltpu.async_copy(
                    hrow_hbm.at[srcall_v.at[pl.ds(coff, CG)]],
                    gbuf_v, sem).wait()
                gbase = sb + coff

                def group(g, cr):
                    for i in range(16):
                        cr = per_edge(masked, gbase, g * 16 + i, cr)
                    return cr

                return lax.fori_loop(0, CG // 16, group, carry)

            carry = proc_chunk(carry, 0, True)
            carry = lax.fori_loop(
                1, nchs - 1,
                lambda c, cr: proc_chunk(cr, c * CG, False), carry)
            carry = lax.fori_loop(
                jnp.maximum(nchs - 1, 1), nchs,
                lambda c, cr: proc_chunk(cr, c * CG, True), carry)
            return carry

        init = (jnp.int32(0), rp_v[pl.ds(16, 16)][0],
                zed, tuple(zed for _ in range(nacc)))
        lax.fori_loop(0, nsuper, super_body, init)
        pltpu.sync_copy(out_v, out_hbm.at[pl.ds(d0 * ch, ROWS * ch)])
        pltpu.sync_copy(den_v, deno_hbm.at[pl.ds(d0 * 16, ROWS * 16)])

    return gat_sc


_gat_sc_256 = _make_gat_sc(256)
_gat_sc_128 = _make_gat_sc(128)


def _make_appnp_sc():
    mesh = plsc.VectorSubcoreMesh(core_axis_name="c", subcore_axis_name="s")

    @functools.partial(
        pl.kernel,
        mesh=mesh,
        out_type=[
            jax.ShapeDtypeStruct((NP_ * H2,), jnp.float32),
            jax.ShapeDtypeStruct((NP_ * H2,), jnp.float32),
        ],
        compiler_params=pltpu.CompilerParams(use_tc_tiling_on_sc=False),
        scratch_types=[
            pltpu.VMEM(((ROWS + 16) * 16,), jnp.int32),
            pltpu.VMEM((ROWS * H2,), jnp.float32),
            pltpu.VMEM((ROWS * 16,), jnp.float32),
            pltpu.VMEM((ECAP,), jnp.int32),
            pltpu.VMEM((ECAP,), jnp.int32),
            pltpu.VMEM((CA, H2), jnp.float32),
            pltpu.VMEM((ROWS * H2,), jnp.float32),
            pltpu.VMEM((ROWS * H2,), jnp.float32),
            pltpu.VMEM((ROWS * H2,), jnp.float32),
            pltpu.SemaphoreType.DMA,
        ],
    )
    def appnp_sc(y_hbm, h_hbm, dis_hbm, rp_hbm, srcp_hbm, dstp_hbm,
                 z_hbm, yo_hbm,
                 rp_v, h_v, dis_v, srcall_v, dstall_v, ybuf_v,
                 acc_v, z_v, yo_v, sem):
        zed = jnp.zeros((16,), jnp.float32)
        wid = lax.axis_index("s") * 2 + lax.axis_index("c")
        d0 = wid * ROWS
        pltpu.sync_copy(rp_hbm.at[pl.ds(d0 * 16, (ROWS + 16) * 16)], rp_v)
        pltpu.sync_copy(h_hbm.at[pl.ds(d0 * H2, ROWS * H2)], h_v)
        pltpu.sync_copy(dis_hbm.at[pl.ds(d0 * 16, ROWS * 16)], dis_v)
        e0 = rp_v[pl.ds(0, 16)][0]
        e1 = rp_v[pl.ds(ROWS * 16, 16)][0]
        base0 = (e0 // CA) * CA
        nsuper = (e1 - base0 + ECAP - 1) // ECAP

        def zero_row(r, _):
            acc_v[pl.ds(r * H2, 16)] = zed
            acc_v[pl.ds(r * H2 + 16, 16)] = zed
            return 0

        lax.fori_loop(0, ROWS, zero_row, 0)

        def per_edge(masked, gbase, gi, dvec, lane):
            dl = dvec[lane] - d0
            y0 = ybuf_v[gi, pl.ds(0, 16)]
            y1 = ybuf_v[gi, pl.ds(16, 16)]
            if masked:
                gidx = gbase + gi
                vm = jnp.logical_and(gidx >= e0, gidx < e1)
                vmv = jnp.broadcast_to(
                    jnp.where(vm, jnp.float32(1.0), jnp.float32(0.0)), (16,))
                y0 = y0 * vmv
                y1 = y1 * vmv
                dl = jnp.where(vm, dl, 0)
            plsc.addupdate(acc_v.at[pl.ds(dl * H2, 16)], y0)
            plsc.addupdate(acc_v.at[pl.ds(dl * H2 + 16, 16)], y1)

        def super_body(s, _):
            sb = base0 + s * ECAP
            pltpu.sync_copy(srcp_hbm.at[pl.ds(sb, ECAP)], srcall_v)
            pltpu.sync_copy(dstp_hbm.at[pl.ds(sb, ECAP)], dstall_v)
            cnt = jnp.minimum(ECAP, e1 - sb)
            nchs = (cnt + CA - 1) // CA

            def proc_chunk(coff, masked):
                pltpu.async_copy(
                    y_hbm.at[srcall_v.at[pl.ds(coff, CA)]],
                    ybuf_v, sem).wait()
                gbase = sb + coff

                def group(g, _):
                    dvec = dstall_v[pl.ds(coff + g * 16, 16)]
                    for i in range(16):
                        per_edge(masked, gbase, g * 16 + i, dvec, i)
                    return 0

                lax.fori_loop(0, CA // 16, group, 0)

            proc_chunk(0, True)
            lax.fori_loop(
                1, nchs - 1,
                lambda c, cr: (proc_chunk(c * CA, False), 0)[1], 0)
            lax.fori_loop(
                jnp.maximum(nchs - 1, 1), nchs,
                lambda c, cr: (proc_chunk(c * CA, True), 0)[1], 0)
            return 0

        lax.fori_loop(0, nsuper, super_body, 0)

        def post_row(r, _):
            sv = jnp.broadcast_to(dis_v[pl.ds(r * 16, 16)][0], (16,))
            a0 = acc_v[pl.ds(r * H2, 16)]
            a1 = acc_v[pl.ds(r * H2 + 16, 16)]
            z0 = (0.9 * sv) * a0 + 0.1 * h_v[pl.ds(r * H2, 16)]
            z1 = (0.9 * sv) * a1 + 0.1 * h_v[pl.ds(r * H2 + 16, 16)]
            z_v[pl.ds(r * H2, 16)] = z0
            z_v[pl.ds(r * H2 + 16, 16)] = z1
            yo_v[pl.ds(r * H2, 16)] = sv * z0
            yo_v[pl.ds(r * H2 + 16, 16)] = sv * z1
            return 0

        lax.fori_loop(0, ROWS, post_row, 0)
        pltpu.sync_copy(z_v, z_hbm.at[pl.ds(d0 * H2, ROWS * H2)])
        pltpu.sync_copy(yo_v, yo_hbm.at[pl.ds(d0 * H2, ROWS * H2)])

    return appnp_sc


_appnp_sc = _make_appnp_sc()


def _pool_final_body(z_ref, aux4_ref, batch_ref, gbc_ref,
                     wgz1_ref, wga1_ref, bg11_ref, wg12_ref, bg12_ref,
                     wnz1_ref, wna1_ref, bn11_ref, wn12_ref, bn12_ref,
                     wgz2_ref, wga2_ref, bg21_ref, wg22_ref, bg22_ref,
                     wnz2_ref, wna2_ref, bn21_ref, wn22_ref, bn22_ref,
                     wgz3_ref, wga3_ref, bg31_ref, wg32_ref, bg32_ref,
                     wnz3_ref, wna3_ref, bn31_ref, wn32_ref, bn32_ref,
                     w3z_ref, w3g_ref, w3a_ref, w3b_ref, b3_ref,
                     w4_ref, b4_ref, w5_ref, b5_ref,
                     out_ref):
    z = z_ref[...]
    aux4 = aux4_ref[...]
    aux3 = aux4[:, 0:3]
    batch = batch_ref[...]
    iota8 = lax.broadcasted_iota(jnp.int32, (1, B), 1)
    mask = (batch == iota8).astype(jnp.float32)

    def pool(wgz, wga, bg1, wg2, bg2, wnz, wna, bn1, wn2, bn2):
        gate = jnp.maximum(z @ wgz + aux3 @ wga + bg1, 0.0) @ wg2 + bg2
        v = jnp.maximum(z @ wnz + aux3 @ wna + bn1, 0.0) @ wn2 + bn2
        gm = jnp.where(mask > 0, gate, -1e30)
        m = jnp.max(gm, axis=0)
        m = jnp.where(m > -1e29, m, 0.0)
        ex = jnp.exp(gate - mask @ m[:, None])
        den = lax.dot_general(mask, ex, (((0,), (0,)), ((), ())))
        a = ex / (mask @ den + 1e-16)
        return lax.dot_general(mask, a * v, (((0,), (0,)), ((), ())))

    g1 = pool(wgz1_ref[...], wga1_ref[...], bg11_ref[...], wg12_ref[...],
              bg12_ref[...], wnz1_ref[...], wna1_ref[...], bn11_ref[...],
              wn12_ref[...], bn12_ref[...])
    g2 = pool(wgz2_ref[...], wga2_ref[...], bg21_ref[...], wg22_ref[...],
              bg22_ref[...], wnz2_ref[...], wna2_ref[...], bn21_ref[...],
              wn22_ref[...], bn22_ref[...])
    g3 = pool(wgz3_ref[...], wga3_ref[...], bg31_ref[...], wg32_ref[...],
              bg32_ref[...], wnz3_ref[...], wna3_ref[...], bn31_ref[...],
              wn32_ref[...], bn32_ref[...])
    g = jnp.concatenate([g1, g2, g3], axis=1)
    tg = g @ w3g_ref[...] + gbc_ref[...] @ w3b_ref[...]
    s = _lrelu(z @ w3z_ref[...] + aux4 @ w3a_ref[...] + mask @ tg
               + b3_ref[...])
    s = _lrelu(s @ w4_ref[...] + b4_ref[...])
    s = s @ w5_ref[...] + b5_ref[...]
    s = jax.nn.sigmoid(s)
    out_ref[...] = lax.dot_general(mask, s, (((0,), (0,)), ((), ())))


def _pool_final(z, aux4, batch, gbc, pool_ws, w3z, w3g, w3a, w3b, b3,
                w4, b4, w5, b5):
    flat = []
    for ws in pool_ws:
        flat.extend(ws)
    return pl.pallas_call(
        _pool_final_body,
        out_shape=jax.ShapeDtypeStruct((B, 1), jnp.float32),
    )(z, aux4, batch, gbc, *flat, w3z, w3g, w3a, w3b, b3, w4, b4, w5, b5)


def _block_diag_att(att_src, att_dst):
    heads, ch = att_src.shape
    eye = jnp.eye(heads, dtype=att_src.dtype)
    a_s = jnp.einsum("hc,hk->hck", att_src, eye).reshape(heads * ch, heads)
    a_d = jnp.einsum("hc,hk->hck", att_dst, eye).reshape(heads * ch, heads)
    return jnp.concatenate([a_s, a_d], axis=1)


def kernel(x, edge_index, batch, J, saved_nodes, infected_nodes,
           size_connected, Omegas, Phis, Lambdas,
           W_gat1, att_src1, att_dst1, b_gat1, W_lin1, b_lin1,
           W_gat2, att_src2, att_dst2, b_gat2, W_lin2, b_lin2,
           Wg11, bg11, Wg12, bg12, Wn11, bn11, Wn12, bn12,
           Wg21, bg21, Wg22, bg22, Wn21, bn21, Wn22, bn22,
           Wg31, bg31, Wg32, bg32, Wn31, bn31, Wn32, bn32,
           W_lin3, b_lin3, W_lin4, b_lin4, W_lin5, b_lin5):
    idt = edge_index.dtype
    loop = jnp.arange(N, dtype=idt)
    src = jnp.concatenate([edge_index[0], loop])
    dst = jnp.concatenate([edge_index[1], loop])
    dst_s, src_s = lax.sort((dst, src), num_keys=1)

    fake_dst = jnp.arange(N, NP_, dtype=idt)
    dst_f = jnp.concatenate([dst_s, fake_dst])
    src_f = jnp.concatenate([src_s, jnp.zeros((NP_ - N,), idt)])
    npad = EP - src_f.shape[0]
    srcp = jnp.concatenate([src_f, jnp.zeros((npad,), idt)])
    dstp = jnp.concatenate([dst_f, jnp.zeros((npad,), idt)])

    rpf = jnp.searchsorted(
        dst_f, jnp.arange(NP_ + 16, dtype=idt)).astype(jnp.int32)
    rp_rep = jnp.repeat(rpf, 16)
    rp = jnp.searchsorted(dst_s, jnp.arange(N + 1, dtype=idt))
    deg = (rp[1:] - rp[:-1]).astype(jnp.float32)[:, None]
    deg = jnp.concatenate([deg, jnp.zeros((NP_ - N, 1), jnp.float32)])

    zrows = jnp.zeros((NP_ - N, 1), jnp.float32)
    x_p = jnp.concatenate([x, jnp.zeros((NP_ - N, IN_DIM), jnp.float32)])
    aux_p = jnp.concatenate([
        jnp.concatenate([J, zrows]), jnp.concatenate([size_connected, zrows]),
        jnp.zeros((NP_, 6), jnp.float32)], axis=1)

    wx = W_gat1[:IN_DIM]
    wjs = jnp.concatenate(
        [W_gat1[IN_DIM:], jnp.zeros((6, HEADS * H1), jnp.float32)])
    asd1 = _block_diag_att(att_src1, att_dst1)
    asd2 = _block_diag_att(att_src2, att_dst2)

    eye4 = jnp.eye(HEADS, dtype=jnp.float32)
    r1 = jnp.repeat(eye4, H1, axis=1)
    r2 = jnp.repeat(eye4, H2, axis=1)

    hrow1, edp1 = _dense1(x_p, aux_p, wx, wjs, asd1)
    out1, den1 = _gat_sc_256(hrow1, edp1.reshape(-1), rp_rep, srcp, dstp)
    hrow2, edp2 = _dense2(out1.reshape(NP_, 256), den1.reshape(NP_, 16), r1,
                          b_gat1, W_lin1, b_lin1, W_gat2, asd2)
    out2, den2 = _gat_sc_128(hrow2, edp2.reshape(-1), rp_rep, srcp, dstp)
    h32, dis16, y = _dense3(out2.reshape(NP_, 128), den2.reshape(NP_, 16), r2,
                            deg, b_gat2, W_lin2, b_lin2)

    h32f = h32.reshape(-1)
    dis16f = dis16.reshape(-1)
    z = None
    for _ in range(K_POWER):
        zf, yf = _appnp_sc(y, h32f, dis16f, rp_rep, srcp, dstp)
        y = yf.reshape(NP_, H2)
        z = zf.reshape(NP_, H2)

    aux4 = jnp.concatenate([J, saved_nodes, infected_nodes, size_connected],
                           axis=1)
    gbc = jnp.concatenate([Omegas, Phis, Lambdas], axis=1)
    pool_ws = [
        (Wg11[:H2], Wg11[H2:], bg11, Wg12, bg12,
         Wn11[:H2], Wn11[H2:], bn11, Wn12, bn12),
        (Wg21[:H2], Wg21[H2:], bg21, Wg22, bg22,
         Wn21[:H2], Wn21[H2:], bn21, Wn22, bn22),
        (Wg31[:H2], Wg31[H2:], bg31, Wg32, bg32,
         Wn31[:H2], Wn31[H2:], bn31, Wn32, bn32),
    ]
    w3z = W_lin3[0:H2]
    w3g = W_lin3[H2:H2 + 3 * H2]
    w3a = W_lin3[H2 + 3 * H2:H2 + 3 * H2 + 4]
    w3b = W_lin3[H2 + 3 * H2 + 4:]
    return _pool_final(z[:N], aux4, batch[:, None], gbc, pool_ws,
                       w3z, w3g, w3a, w3b, b_lin3,
                       W_lin4, b_lin4, W_lin5, b_lin5)


# speedup vs baseline: 22.5451x; 1.0982x over previous
import functools

import jax
import jax.numpy as jnp
from jax import lax
from jax.experimental import pallas as pl
from jax.experimental.pallas import tpu as pltpu
from jax.experimental.pallas import tpu_sc as plsc

N = 10000
E = 160000
B = 8
IN_DIM = 128
H1 = 64
H2 = 32
HEADS = 4
K_POWER = 10
ALPHA = 0.1

NP_ = 10240
TILES = 32
ROWS = NP_ // TILES
ECAP = 6144
EP = 170240 + ECAP
CG = 64
CA = 128
BLK = 1024


def _lrelu(t):
    return jnp.where(t >= 0, t, 0.2 * t)


def _dense1_body(x_ref, aux_ref, wx_ref, wjs_ref, asd_ref, hrow_ref, ed_ref):
    h = x_ref[...] @ wx_ref[...] + aux_ref[...] @ wjs_ref[...]
    esed = h @ asd_ref[...]
    zpad = jnp.zeros((h.shape[0], 12), jnp.float32)
    hrow_ref[...] = jnp.concatenate([h, esed[:, 0:4], zpad], axis=1)
    ed_ref[...] = jnp.concatenate([esed[:, 4:8], zpad], axis=1)


def _dense1(x_p, aux_p, wx, wjs, asd):
    g = NP_ // BLK
    return pl.pallas_call(
        _dense1_body,
        grid=(g,),
        in_specs=[
            pl.BlockSpec((BLK, IN_DIM), lambda i: (i, 0)),
            pl.BlockSpec((BLK, 8), lambda i: (i, 0)),
            pl.BlockSpec(wx.shape, lambda i: (0, 0)),
            pl.BlockSpec(wjs.shape, lambda i: (0, 0)),
            pl.BlockSpec(asd.shape, lambda i: (0, 0)),
        ],
        out_specs=[
            pl.BlockSpec((BLK, 272), lambda i: (i, 0)),
            pl.BlockSpec((BLK, 16), lambda i: (i, 0)),
        ],
        out_shape=[
            jax.ShapeDtypeStruct((NP_, 272), jnp.float32),
            jax.ShapeDtypeStruct((NP_, 16), jnp.float32),
        ],
    )(x_p, aux_p, wx, wjs, asd)


def _dense2_body(o_ref, den_ref, r1_ref, bg_ref, wl_ref, bl_ref, wg_ref,
                 asd_ref, hrow_ref, ed_ref):
    inv4 = 1.0 / (den_ref[...][:, 0:4] + 1e-16)
    o = o_ref[...] * (inv4 @ r1_ref[...])
    t = _lrelu((o + bg_ref[...]) @ wl_ref[...] + bl_ref[...])
    h2 = t @ wg_ref[...]
    esed = h2 @ asd_ref[...]
    zpad = jnp.zeros((h2.shape[0], 12), jnp.float32)
    hrow_ref[...] = jnp.concatenate([h2, esed[:, 0:4], zpad], axis=1)
    ed_ref[...] = jnp.concatenate([esed[:, 4:8], zpad], axis=1)


def _dense2(out1, den1, r1, b_gat1, W_lin1, b_lin1, W_gat2, asd2):
    g = NP_ // BLK
    return pl.pallas_call(
        _dense2_body,
        grid=(g,),
        in_specs=[
            pl.BlockSpec((BLK, 256), lambda i: (i, 0)),
            pl.BlockSpec((BLK, 16), lambda i: (i, 0)),
            pl.BlockSpec((HEADS, 256), lambda i: (0, 0)),
            pl.BlockSpec((1, 256), lambda i: (0, 0)),
            pl.BlockSpec(W_lin1.shape, lambda i: (0, 0)),
            pl.BlockSpec((1, H1), lambda i: (0, 0)),
            pl.BlockSpec(W_gat2.shape, lambda i: (0, 0)),
            pl.BlockSpec(asd2.shape, lambda i: (0, 0)),
        ],
        out_specs=[
            pl.BlockSpec((BLK, 144), lambda i: (i, 0)),
            pl.BlockSpec((BLK, 16), lambda i: (i, 0)),
        ],
        out_shape=[
            jax.ShapeDtypeStruct((NP_, 144), jnp.float32),
            jax.ShapeDtypeStruct((NP_, 16), jnp.float32),
        ],
    )(out1, den1, r1, b_gat1[None, :], W_lin1, b_lin1[None, :], W_gat2, asd2)


def _dense3_body(o_ref, den_ref, r2_ref, deg_ref, bg_ref, wl_ref, bl_ref,
                 h_ref, dis_ref, y_ref):
    inv4 = 1.0 / (den_ref[...][:, 0:4] + 1e-16)
    o = o_ref[...] * (inv4 @ r2_ref[...])
    h32 = _lrelu((o + bg_ref[...]) @ wl_ref[...] + bl_ref[...])
    deg = deg_ref[...]
    dis = jnp.where(deg > 0, lax.rsqrt(jnp.maximum(deg, 1e-12)), 0.0)
    h_ref[...] = h32
    dis_ref[...] = jnp.concatenate(
        [dis, jnp.zeros((dis.shape[0], 15), jnp.float32)], axis=1)
    y_ref[...] = h32 * dis


def _dense3(out2, den2, r2, deg, b_gat2, W_lin2, b_lin2):
    g = NP_ // BLK
    return pl.pallas_call(
        _dense3_body,
        grid=(g,),
        in_specs=[
            pl.BlockSpec((BLK, 128), lambda i: (i, 0)),
            pl.BlockSpec((BLK, 16), lambda i: (i, 0)),
            pl.BlockSpec((HEADS, 128), lambda i: (0, 0)),
            pl.BlockSpec((BLK, 1), lambda i: (i, 0)),
            pl.BlockSpec((1, 128), lambda i: (0, 0)),
            pl.BlockSpec(W_lin2.shape, lambda i: (0, 0)),
            pl.BlockSpec((1, H2), lambda i: (0, 0)),
        ],
        out_specs=[
            pl.BlockSpec((BLK, H2), lambda i: (i, 0)),
            pl.BlockSpec((BLK, 16), lambda i: (i, 0)),
            pl.BlockSpec((BLK, H2), lambda i: (i, 0)),
        ],
        out_shape=[
            jax.ShapeDtypeStruct((NP_, H2), jnp.float32),
            jax.ShapeDtypeStruct((NP_, 16), jnp.float32),
            jax.ShapeDtypeStruct((NP_, H2), jnp.float32),
        ],
    )(out2, den2, r2, deg, b_gat2[None, :], W_lin2, b_lin2[None, :])


def _bcast_lane(v, i):
    return jnp.broadcast_to(v[i], (16,))


def _make_gat_sc(ch):
    roww = ch + 16
    cph = ch // HEADS
    sub = cph // 16
    nacc = HEADS * sub
    mesh = plsc.VectorSubcoreMesh(core_axis_name="c", subcore_axis_name="s")

    @functools.partial(
        pl.kernel,
        mesh=mesh,
        out_type=[
            jax.ShapeDtypeStruct((NP_ * ch,), jnp.float32),
            jax.ShapeDtypeStruct((NP_ * 16,), jnp.float32),
        ],
        compiler_params=pltpu.CompilerParams(use_tc_tiling_on_sc=False),
        scratch_types=[
            pltpu.VMEM(((ROWS + 16) * 16,), jnp.int32),
            pltpu.VMEM((ROWS * 16,), jnp.float32),
            pltpu.VMEM((ECAP,), jnp.int32),
            pltpu.VMEM((CG, roww), jnp.float32),
            pltpu.VMEM((ROWS * ch,), jnp.float32),
            pltpu.VMEM((ROWS * 16,), jnp.float32),
            pltpu.SemaphoreType.DMA,
        ],
    )
    def gat_sc(hrow_hbm, edp_hbm, rp_hbm, srcp_hbm, dstp_hbm,
               out_hbm, deno_hbm,
               rp_v, ed_v, srcall_v, gbuf_v, out_v, den_v, sem):
        zed = jnp.zeros((16,), jnp.float32)
        wid = lax.axis_index("s") * 2 + lax.axis_index("c")
        d0 = wid * ROWS
        pltpu.sync_copy(rp_hbm.at[pl.ds(d0 * 16, (ROWS + 16) * 16)], rp_v)
        pltpu.sync_copy(edp_hbm.at[pl.ds(d0 * 16, ROWS * 16)], ed_v)
        e0 = rp_v[pl.ds(0, 16)][0]
        e1 = rp_v[pl.ds(ROWS * 16, 16)][0]
        base0 = (e0 // CG) * CG
        nsuper = (e1 - base0 + ECAP - 1) // ECAP

        def per_edge(masked, gbase, gi, carry):
            r, row_end, den, accs = carry
            gidx = gbase + gi
            esv = gbuf_v[gi, pl.ds(ch, 16)]
            edrow = ed_v[pl.ds(r * 16, 16)]
            ex = jnp.exp(_lrelu(esv + edrow))
            if masked:
                vm = jnp.logical_and(gidx >= e0, gidx < e1)
                ex = jnp.where(vm, ex, zed)
            den = den + ex
            naccs = []
            for h in range(HEADS):
                w = _bcast_lane(ex, h)
                for j in range(sub):
                    naccs.append(accs[h * sub + j]
                                 + w * gbuf_v[gi, pl.ds(h * cph + j * 16, 16)])
            is_end = (gidx + 1) == row_end
            if masked:
                is_end = jnp.logical_and(is_end, vm)

            def flush(den_, accs_):
                den_v[pl.ds(r * 16, 16)] = den_
                for k in range(nacc):
                    out_v[pl.ds(r * ch + k * 16, 16)] = accs_[k]
                r2 = r + 1
                row_end2 = rp_v[pl.ds((r2 + 1) * 16, 16)][0]
                return (r2, row_end2, zed,
                        tuple(zed for _ in range(nacc)))

            def keep(den_, accs_):
                return (r, row_end, den_, accs_)

            return lax.cond(is_end, flush, keep, den, tuple(naccs))

        def super_body(s, carry):
            sb = base0 + s * ECAP
            pltpu.sync_copy(srcp_hbm.at[pl.ds(sb, ECAP)], srcall_v)
            cnt = jnp.minimum(ECAP, e1 - sb)
            nchs = (cnt + CG - 1) // CG

            def proc_chunk(carry, coff, masked):
                pltpu.async_copy(
                    hrow_hbm.at[srcall_v.at[pl.ds(coff, CG)]],
                    gbuf_v, sem).wait()
                gbase = sb + coff

                def group(g, cr):
                    for i in range(16):
                        cr = per_edge(masked, gbase, g * 16 + i, cr)
                    return cr

                return lax.fori_loop(0, CG // 16, group, carry)

            carry = proc_chunk(carry, 0, True)
            carry = lax.fori_loop(
                1, nchs - 1,
                lambda c, cr: proc_chunk(cr, c * CG, False), carry)
            carry = lax.fori_loop(
                jnp.maximum(nchs - 1, 1), nchs,
                lambda c, cr: proc_chunk(cr, c * CG, True), carry)
            return carry

        init = (jnp.int32(0), rp_v[pl.ds(16, 16)][0],
                zed, tuple(zed for _ in range(nacc)))
        lax.fori_loop(0, nsuper, super_body, init)
        pltpu.sync_copy(out_v, out_hbm.at[pl.ds(d0 * ch, ROWS * ch)])
        pltpu.sync_copy(den_v, deno_hbm.at[pl.ds(d0 * 16, ROWS * 16)])

    return gat_sc


_gat_sc_256 = _make_gat_sc(256)
_gat_sc_128 = _make_gat_sc(128)


def _make_appnp_sc():
    mesh = plsc.VectorSubcoreMesh(core_axis_name="c", subcore_axis_name="s")

    @functools.partial(
        pl.kernel,
        mesh=mesh,
        out_type=[
            jax.ShapeDtypeStruct((NP_ * H2,), jnp.float32),
            jax.ShapeDtypeStruct((NP_ * H2,), jnp.float32),
        ],
        compiler_params=pltpu.CompilerParams(use_tc_tiling_on_sc=False),
        scratch_types=[
            pltpu.VMEM(((ROWS + 16) * 16,), jnp.int32),
            pltpu.VMEM((ROWS * H2,), jnp.float32),
            pltpu.VMEM((ROWS * 16,), jnp.float32),
            pltpu.VMEM((ECAP,), jnp.int32),
            pltpu.VMEM((ECAP,), jnp.int32),
            pltpu.VMEM((CA, H2), jnp.float32),
            pltpu.VMEM((CA, H2), jnp.float32),
            pltpu.VMEM((ROWS * H2,), jnp.float32),
            pltpu.VMEM((ROWS * H2,), jnp.float32),
            pltpu.VMEM((ROWS * H2,), jnp.float32),
            pltpu.SemaphoreType.DMA,
            pltpu.SemaphoreType.DMA,
        ],
    )
    def appnp_sc(y_hbm, h_hbm, dis_hbm, rp_hbm, srcp_hbm, dstp_hbm,
                 z_hbm, yo_hbm,
                 rp_v, h_v, dis_v, srcall_v, dstall_v, ybuf_v, ybuf2_v,
                 acc_v, z_v, yo_v, sem, sem2):
        zed = jnp.zeros((16,), jnp.float32)
        wid = lax.axis_index("s") * 2 + lax.axis_index("c")
        d0 = wid * ROWS
        pltpu.sync_copy(rp_hbm.at[pl.ds(d0 * 16, (ROWS + 16) * 16)], rp_v)
        pltpu.sync_copy(h_hbm.at[pl.ds(d0 * H2, ROWS * H2)], h_v)
        pltpu.sync_copy(dis_hbm.at[pl.ds(d0 * 16, ROWS * 16)], dis_v)
        e0 = rp_v[pl.ds(0, 16)][0]
        e1 = rp_v[pl.ds(ROWS * 16, 16)][0]
        base0 = (e0 // CA) * CA
        nsuper = (e1 - base0 + ECAP - 1) // ECAP

        def zero_row(r, _):
            acc_v[pl.ds(r * H2, 16)] = zed
            acc_v[pl.ds(r * H2 + 16, 16)] = zed
            return 0

        lax.fori_loop(0, ROWS, zero_row, 0)

        def per_edge(masked, gbase, gi, dvec, lane, buf):
            dl = dvec[lane] - d0
            y0 = buf[gi, pl.ds(0, 16)]
            y1 = buf[gi, pl.ds(16, 16)]
            if masked:
                gidx = gbase + gi
                vm = jnp.logical_and(gidx >= e0, gidx < e1)
                vmv = jnp.broadcast_to(
                    jnp.where(vm, jnp.float32(1.0), jnp.float32(0.0)), (16,))
                y0 = y0 * vmv
                y1 = y1 * vmv
                dl = jnp.where(vm, dl, 0)
            plsc.addupdate(acc_v.at[pl.ds(dl * H2, 16)], y0)
            plsc.addupdate(acc_v.at[pl.ds(dl * H2 + 16, 16)], y1)

        def super_body(s, _):
            sb = base0 + s * ECAP
            pltpu.sync_copy(srcp_hbm.at[pl.ds(sb, ECAP)], srcall_v)
            pltpu.sync_copy(dstp_hbm.at[pl.ds(sb, ECAP)], dstall_v)
            cnt = jnp.minimum(ECAP, e1 - sb)
            nchs = (cnt + CA - 1) // CA

            def issue(coff, buf, sm):
                pltpu.async_copy(
                    y_hbm.at[srcall_v.at[pl.ds(coff, CA)]], buf, sm)

            def drain(buf, sm):
                pltpu.make_async_copy(y_hbm.at[pl.ds(0, CA)], buf, sm).wait()

            def process(coff, buf, masked):
                gbase = sb + coff

                def group(g, _):
                    dvec = dstall_v[pl.ds(coff + g * 16, 16)]
                    for i in range(16):
                        per_edge(masked, gbase, g * 16 + i, dvec, i,
                                 buf)
                    return 0

                lax.fori_loop(0, CA // 16, group, 0)

            def proc_chunk(coff, masked):
                issue(coff, ybuf_v, sem)
                drain(ybuf_v, sem)
                process(coff, ybuf_v, masked)

            proc_chunk(0, True)
            hi = nchs - 1

            @pl.when(hi > 1)
            def _prologue():
                issue(1 * CA, ybuf_v, sem)

            def pair(p, _):
                c0 = 1 + 2 * p

                @pl.when(c0 + 1 < hi)
                def _i1():
                    issue((c0 + 1) * CA, ybuf2_v, sem2)

                drain(ybuf_v, sem)
                process(c0 * CA, ybuf_v, False)

                @pl.when(c0 + 2 < hi)
                def _i2():
                    issue((c0 + 2) * CA, ybuf_v, sem)

                @pl.when(c0 + 1 < hi)
                def _p2():
                    drain(ybuf2_v, sem2)
                    process((c0 + 1) * CA, ybuf2_v, False)

                return 0

            npair = jnp.maximum(hi - 1, 0) // 2 + jnp.maximum(hi - 1, 0) % 2
            lax.fori_loop(0, npair, pair, 0)
            lax.fori_loop(
                jnp.maximum(nchs - 1, 1), nchs,
                lambda c, cr: (proc_chunk(c * CA, True), 0)[1], 0)
            return 0

        lax.fori_loop(0, nsuper, super_body, 0)

        def post_row(r, _):
            sv = jnp.broadcast_to(dis_v[pl.ds(r * 16, 16)][0], (16,))
            a0 = acc_v[pl.ds(r * H2, 16)]
            a1 = acc_v[pl.ds(r * H2 + 16, 16)]
            z0 = (0.9 * sv) * a0 + 0.1 * h_v[pl.ds(r * H2, 16)]
            z1 = (0.9 * sv) * a1 + 0.1 * h_v[pl.ds(r * H2 + 16, 16)]
            z_v[pl.ds(r * H2, 16)] = z0
            z_v[pl.ds(r * H2 + 16, 16)] = z1
            yo_v[pl.ds(r * H2, 16)] = sv * z0
            yo_v[pl.ds(r * H2 + 16, 16)] = sv * z1
            return 0

        lax.fori_loop(0, ROWS, post_row, 0)
        pltpu.sync_copy(z_v, z_hbm.at[pl.ds(d0 * H2, ROWS * H2)])
        pltpu.sync_copy(yo_v, yo_hbm.at[pl.ds(d0 * H2, ROWS * H2)])

    return appnp_sc


_appnp_sc = _make_appnp_sc()


def _pool_final_body(z_ref, aux4_ref, batch_ref, gbc_ref,
                     wgz1_ref, wga1_ref, bg11_ref, wg12_ref, bg12_ref,
                     wnz1_ref, wna1_ref, bn11_ref, wn12_ref, bn12_ref,
                     wgz2_ref, wga2_ref, bg21_ref, wg22_ref, bg22_ref,
                     wnz2_ref, wna2_ref, bn21_ref, wn22_ref, bn22_ref,
                     wgz3_ref, wga3_ref, bg31_ref, wg32_ref, bg32_ref,
                     wnz3_ref, wna3_ref, bn31_ref, wn32_ref, bn32_ref,
                     w3z_ref, w3g_ref, w3a_ref, w3b_ref, b3_ref,
                     w4_ref, b4_ref, w5_ref, b5_ref,
                     out_ref):
    z = z_ref[...]
    aux4 = aux4_ref[...]
    aux3 = aux4[:, 0:3]
    batch = batch_ref[...]
    iota8 = lax.broadcasted_iota(jnp.int32, (1, B), 1)
    mask = (batch == iota8).astype(jnp.float32)

    def pool(wgz, wga, bg1, wg2, bg2, wnz, wna, bn1, wn2, bn2):
        gate = jnp.maximum(z @ wgz + aux3 @ wga + bg1, 0.0) @ wg2 + bg2
        v = jnp.maximum(z @ wnz + aux3 @ wna + bn1, 0.0) @ wn2 + bn2
        gm = jnp.where(mask > 0, gate, -1e30)
        m = jnp.max(gm, axis=0)
        m = jnp.where(m > -1e29, m, 0.0)
        ex = jnp.exp(gate - mask @ m[:, None])
        den = lax.dot_general(mask, ex, (((0,), (0,)), ((), ())))
        a = ex / (mask @ den + 1e-16)
        return lax.dot_general(mask, a * v, (((0,), (0,)), ((), ())))

    g1 = pool(wgz1_ref[...], wga1_ref[...], bg11_ref[...], wg12_ref[...],
              bg12_ref[...], wnz1_ref[...], wna1_ref[...], bn11_ref[...],
              wn12_ref[...], bn12_ref[...])
    g2 = pool(wgz2_ref[...], wga2_ref[...], bg21_ref[...], wg22_ref[...],
              bg22_ref[...], wnz2_ref[...], wna2_ref[...], bn21_ref[...],
              wn22_ref[...], bn22_ref[...])
    g3 = pool(wgz3_ref[...], wga3_ref[...], bg31_ref[...], wg32_ref[...],
              bg32_ref[...], wnz3_ref[...], wna3_ref[...], bn31_ref[...],
              wn32_ref[...], bn32_ref[...])
    g = jnp.concatenate([g1, g2, g3], axis=1)
    tg = g @ w3g_ref[...] + gbc_ref[...] @ w3b_ref[...]
    s = _lrelu(z @ w3z_ref[...] + aux4 @ w3a_ref[...] + mask @ tg
               + b3_ref[...])
    s = _lrelu(s @ w4_ref[...] + b4_ref[...])
    s = s @ w5_ref[...] + b5_ref[...]
    s = jax.nn.sigmoid(s)
    out_ref[...] = lax.dot_general(mask, s, (((0,), (0,)), ((), ())))


def _pool_final(z, aux4, batch, gbc, pool_ws, w3z, w3g, w3a, w3b, b3,
                w4, b4, w5, b5):
    flat = []
    for ws in pool_ws:
        flat.extend(ws)
    return pl.pallas_call(
        _pool_final_body,
        out_shape=jax.ShapeDtypeStruct((B, 1), jnp.float32),
    )(z, aux4, batch, gbc, *flat, w3z, w3g, w3a, w3b, b3, w4, b4, w5, b5)


def _block_diag_att(att_src, att_dst):
    heads, ch = att_src.shape
    eye = jnp.eye(heads, dtype=att_src.dtype)
    a_s = jnp.einsum("hc,hk->hck", att_src, eye).reshape(heads * ch, heads)
    a_d = jnp.einsum("hc,hk->hck", att_dst, eye).reshape(heads * ch, heads)
    return jnp.concatenate([a_s, a_d], axis=1)


def kernel(x, edge_index, batch, J, saved_nodes, infected_nodes,
           size_connected, Omegas, Phis, Lambdas,
           W_gat1, att_src1, att_dst1, b_gat1, W_lin1, b_lin1,
           W_gat2, att_src2, att_dst2, b_gat2, W_lin2, b_lin2,
           Wg11, bg11, Wg12, bg12, Wn11, bn11, Wn12, bn12,
           Wg21, bg21, Wg22, bg22, Wn21, bn21, Wn22, bn22,
           Wg31, bg31, Wg32, bg32, Wn31, bn31, Wn32, bn32,
           W_lin3, b_lin3, W_lin4, b_lin4, W_lin5, b_lin5):
    idt = edge_index.dtype
    loop = jnp.arange(N, dtype=idt)
    src = jnp.concatenate([edge_index[0], loop])
    dst = jnp.concatenate([edge_index[1], loop])
    dst_s, src_s = lax.sort((dst, src), num_keys=1)

    fake_dst = jnp.arange(N, NP_, dtype=idt)
    dst_f = jnp.concatenate([dst_s, fake_dst])
    src_f = jnp.concatenate([src_s, jnp.zeros((NP_ - N,), idt)])
    npad = EP - src_f.shape[0]
    srcp = jnp.concatenate([src_f, jnp.zeros((npad,), idt)])
    dstp = jnp.concatenate([dst_f, jnp.zeros((npad,), idt)])

    rpf = jnp.searchsorted(
        dst_f, jnp.arange(NP_ + 16, dtype=idt)).astype(jnp.int32)
    rp_rep = jnp.repeat(rpf, 16)
    rp = jnp.searchsorted(dst_s, jnp.arange(N + 1, dtype=idt))
    deg = (rp[1:] - rp[:-1]).astype(jnp.float32)[:, None]
    deg = jnp.concatenate([deg, jnp.zeros((NP_ - N, 1), jnp.float32)])

    zrows = jnp.zeros((NP_ - N, 1), jnp.float32)
    x_p = jnp.concatenate([x, jnp.zeros((NP_ - N, IN_DIM), jnp.float32)])
    aux_p = jnp.concatenate([
        jnp.concatenate([J, zrows]), jnp.concatenate([size_connected, zrows]),
        jnp.zeros((NP_, 6), jnp.float32)], axis=1)

    wx = W_gat1[:IN_DIM]
    wjs = jnp.concatenate(
        [W_gat1[IN_DIM:], jnp.zeros((6, HEADS * H1), jnp.float32)])
    asd1 = _block_diag_att(att_src1, att_dst1)
    asd2 = _block_diag_att(att_src2, att_dst2)

    eye4 = jnp.eye(HEADS, dtype=jnp.float32)
    r1 = jnp.repeat(eye4, H1, axis=1)
    r2 = jnp.repeat(eye4, H2, axis=1)

    hrow1, edp1 = _dense1(x_p, aux_p, wx, wjs, asd1)
    out1, den1 = _gat_sc_256(hrow1, edp1.reshape(-1), rp_rep, srcp, dstp)
    hrow2, edp2 = _dense2(out1.reshape(NP_, 256), den1.reshape(NP_, 16), r1,
                          b_gat1, W_lin1, b_lin1, W_gat2, asd2)
    out2, den2 = _gat_sc_128(hrow2, edp2.reshape(-1), rp_rep, srcp, dstp)
    h32, dis16, y = _dense3(out2.reshape(NP_, 128), den2.reshape(NP_, 16), r2,
                            deg, b_gat2, W_lin2, b_lin2)

    h32f = h32.reshape(-1)
    dis16f = dis16.reshape(-1)
    z = None
    for _ in range(K_POWER):
        zf, yf = _appnp_sc(y, h32f, dis16f, rp_rep, srcp, dstp)
        y = yf.reshape(NP_, H2)
        z = zf.reshape(NP_, H2)

    aux4 = jnp.concatenate([J, saved_nodes, infected_nodes, size_connected],
                           axis=1)
    gbc = jnp.concatenate([Omegas, Phis, Lambdas], axis=1)
    pool_ws = [
        (Wg11[:H2], Wg11[H2:], bg11, Wg12, bg12,
         Wn11[:H2], Wn11[H2:], bn11, Wn12, bn12),
        (Wg21[:H2], Wg21[H2:], bg21, Wg22, bg22,
         Wn21[:H2], Wn21[H2:], bn21, Wn22, bn22),
        (Wg31[:H2], Wg31[H2:], bg31, Wg32, bg32,
         Wn31[:H2], Wn31[H2:], bn31, Wn32, bn32),
    ]
    w3z = W_lin3[0:H2]
    w3g = W_lin3[H2:H2 + 3 * H2]
    w3a = W_lin3[H2 + 3 * H2:H2 + 3 * H2 + 4]
    w3b = W_lin3[H2 + 3 * H2 + 4:]
    return _pool_final(z[:N], aux4, batch[:, None], gbc, pool_ws,
                       w3z, w3g, w3a, w3b, b_lin3,
                       W_lin4, b_lin4, W_lin5, b_lin5)


# speedup vs baseline: 23.5244x; 1.0434x over previous
import functools

import jax
import jax.numpy as jnp
from jax import lax
from jax.experimental import pallas as pl
from jax.experimental.pallas import tpu as pltpu
from jax.experimental.pallas import tpu_sc as plsc

N = 10000
E = 160000
B = 8
IN_DIM = 128
H1 = 64
H2 = 32
HEADS = 4
K_POWER = 10
ALPHA = 0.1

NP_ = 10240
TILES = 32
ROWS = NP_ // TILES
ECAP = 6144
EP = 170240 + ECAP
CG = 48
CA = 128
BLK = 1024


def _lrelu(t):
    return jnp.where(t >= 0, t, 0.2 * t)


def _dense1_body(x_ref, aux_ref, wx_ref, wjs_ref, asd_ref, hrow_ref, ed_ref):
    h = x_ref[...] @ wx_ref[...] + aux_ref[...] @ wjs_ref[...]
    esed = h @ asd_ref[...]
    zpad = jnp.zeros((h.shape[0], 12), jnp.float32)
    hrow_ref[...] = jnp.concatenate([h, esed[:, 0:4], zpad], axis=1)
    ed_ref[...] = jnp.concatenate([esed[:, 4:8], zpad], axis=1)


def _dense1(x_p, aux_p, wx, wjs, asd):
    g = NP_ // BLK
    return pl.pallas_call(
        _dense1_body,
        grid=(g,),
        in_specs=[
            pl.BlockSpec((BLK, IN_DIM), lambda i: (i, 0)),
            pl.BlockSpec((BLK, 8), lambda i: (i, 0)),
            pl.BlockSpec(wx.shape, lambda i: (0, 0)),
            pl.BlockSpec(wjs.shape, lambda i: (0, 0)),
            pl.BlockSpec(asd.shape, lambda i: (0, 0)),
        ],
        out_specs=[
            pl.BlockSpec((BLK, 272), lambda i: (i, 0)),
            pl.BlockSpec((BLK, 16), lambda i: (i, 0)),
        ],
        out_shape=[
            jax.ShapeDtypeStruct((NP_, 272), jnp.float32),
            jax.ShapeDtypeStruct((NP_, 16), jnp.float32),
        ],
    )(x_p, aux_p, wx, wjs, asd)


def _dense2_body(o_ref, den_ref, r1_ref, bg_ref, wl_ref, bl_ref, wg_ref,
                 asd_ref, hrow_ref, ed_ref):
    inv4 = 1.0 / (den_ref[...][:, 0:4] + 1e-16)
    o = o_ref[...] * (inv4 @ r1_ref[...])
    t = _lrelu((o + bg_ref[...]) @ wl_ref[...] + bl_ref[...])
    h2 = t @ wg_ref[...]
    esed = h2 @ asd_ref[...]
    zpad = jnp.zeros((h2.shape[0], 12), jnp.float32)
    hrow_ref[...] = jnp.concatenate([h2, esed[:, 0:4], zpad], axis=1)
    ed_ref[...] = jnp.concatenate([esed[:, 4:8], zpad], axis=1)


def _dense2(out1, den1, r1, b_gat1, W_lin1, b_lin1, W_gat2, asd2):
    g = NP_ // BLK
    return pl.pallas_call(
        _dense2_body,
        grid=(g,),
        in_specs=[
            pl.BlockSpec((BLK, 256), lambda i: (i, 0)),
            pl.BlockSpec((BLK, 16), lambda i: (i, 0)),
            pl.BlockSpec((HEADS, 256), lambda i: (0, 0)),
            pl.BlockSpec((1, 256), lambda i: (0, 0)),
            pl.BlockSpec(W_lin1.shape, lambda i: (0, 0)),
            pl.BlockSpec((1, H1), lambda i: (0, 0)),
            pl.BlockSpec(W_gat2.shape, lambda i: (0, 0)),
            pl.BlockSpec(asd2.shape, lambda i: (0, 0)),
        ],
        out_specs=[
            pl.BlockSpec((BLK, 144), lambda i: (i, 0)),
            pl.BlockSpec((BLK, 16), lambda i: (i, 0)),
        ],
        out_shape=[
            jax.ShapeDtypeStruct((NP_, 144), jnp.float32),
            jax.ShapeDtypeStruct((NP_, 16), jnp.float32),
        ],
    )(out1, den1, r1, b_gat1[None, :], W_lin1, b_lin1[None, :], W_gat2, asd2)


def _dense3_body(o_ref, den_ref, r2_ref, deg_ref, bg_ref, wl_ref, bl_ref,
                 h_ref, dis_ref, y_ref):
    inv4 = 1.0 / (den_ref[...][:, 0:4] + 1e-16)
    o = o_ref[...] * (inv4 @ r2_ref[...])
    h32 = _lrelu((o + bg_ref[...]) @ wl_ref[...] + bl_ref[...])
    deg = deg_ref[...]
    dis = jnp.where(deg > 0, lax.rsqrt(jnp.maximum(deg, 1e-12)), 0.0)
    h_ref[...] = h32
    dis_ref[...] = jnp.concatenate(
        [dis, jnp.zeros((dis.shape[0], 15), jnp.float32)], axis=1)
    y_ref[...] = h32 * dis


def _dense3(out2, den2, r2, deg, b_gat2, W_lin2, b_lin2):
    g = NP_ // BLK
    return pl.pallas_call(
        _dense3_body,
        grid=(g,),
        in_specs=[
            pl.BlockSpec((BLK, 128), lambda i: (i, 0)),
            pl.BlockSpec((BLK, 16), lambda i: (i, 0)),
            pl.BlockSpec((HEADS, 128), lambda i: (0, 0)),
            pl.BlockSpec((BLK, 1), lambda i: (i, 0)),
            pl.BlockSpec((1, 128), lambda i: (0, 0)),
            pl.BlockSpec(W_lin2.shape, lambda i: (0, 0)),
            pl.BlockSpec((1, H2), lambda i: (0, 0)),
        ],
        out_specs=[
            pl.BlockSpec((BLK, H2), lambda i: (i, 0)),
            pl.BlockSpec((BLK, 16), lambda i: (i, 0)),
            pl.BlockSpec((BLK, H2), lambda i: (i, 0)),
        ],
        out_shape=[
            jax.ShapeDtypeStruct((NP_, H2), jnp.float32),
            jax.ShapeDtypeStruct((NP_, 16), jnp.float32),
            jax.ShapeDtypeStruct((NP_, H2), jnp.float32),
        ],
    )(out2, den2, r2, deg, b_gat2[None, :], W_lin2, b_lin2[None, :])


def _bcast_lane(v, i):
    return jnp.broadcast_to(v[i], (16,))


def _make_gat_sc(ch):
    roww = ch + 16
    cph = ch // HEADS
    sub = cph // 16
    nacc = HEADS * sub
    mesh = plsc.VectorSubcoreMesh(core_axis_name="c", subcore_axis_name="s")

    @functools.partial(
        pl.kernel,
        mesh=mesh,
        out_type=[
            jax.ShapeDtypeStruct((NP_ * ch,), jnp.float32),
            jax.ShapeDtypeStruct((NP_ * 16,), jnp.float32),
        ],
        compiler_params=pltpu.CompilerParams(use_tc_tiling_on_sc=False),
        scratch_types=[
            pltpu.VMEM(((ROWS + 16) * 16,), jnp.int32),
            pltpu.VMEM((ROWS * 16,), jnp.float32),
            pltpu.VMEM((ECAP,), jnp.int32),
            pltpu.VMEM((CG, roww), jnp.float32),
            pltpu.VMEM((CG, roww), jnp.float32),
            pltpu.VMEM((ROWS * ch,), jnp.float32),
            pltpu.VMEM((ROWS * 16,), jnp.float32),
            pltpu.SemaphoreType.DMA,
            pltpu.SemaphoreType.DMA,
        ],
    )
    def gat_sc(hrow_hbm, edp_hbm, rp_hbm, srcp_hbm, dstp_hbm,
               out_hbm, deno_hbm,
               rp_v, ed_v, srcall_v, gbuf_v, gbuf2_v, out_v, den_v,
               sem, sem2):
        zed = jnp.zeros((16,), jnp.float32)
        wid = lax.axis_index("s") * 2 + lax.axis_index("c")
        d0 = wid * ROWS
        pltpu.sync_copy(rp_hbm.at[pl.ds(d0 * 16, (ROWS + 16) * 16)], rp_v)
        pltpu.sync_copy(edp_hbm.at[pl.ds(d0 * 16, ROWS * 16)], ed_v)
        e0 = rp_v[pl.ds(0, 16)][0]
        e1 = rp_v[pl.ds(ROWS * 16, 16)][0]
        base0 = (e0 // CG) * CG
        nsuper = (e1 - base0 + ECAP - 1) // ECAP

        def per_edge(masked, gbase, gi, carry, buf):
            r, row_end, den, accs = carry
            gidx = gbase + gi
            esv = buf[gi, pl.ds(ch, 16)]
            edrow = ed_v[pl.ds(r * 16, 16)]
            ex = jnp.exp(_lrelu(esv + edrow))
            if masked:
                vm = jnp.logical_and(gidx >= e0, gidx < e1)
                ex = jnp.where(vm, ex, zed)
            den = den + ex
            naccs = []
            for h in range(HEADS):
                w = _bcast_lane(ex, h)
                for j in range(sub):
                    naccs.append(accs[h * sub + j]
                                 + w * buf[gi, pl.ds(h * cph + j * 16, 16)])
            is_end = (gidx + 1) == row_end
            if masked:
                is_end = jnp.logical_and(is_end, vm)

            def flush(den_, accs_):
                den_v[pl.ds(r * 16, 16)] = den_
                for k in range(nacc):
                    out_v[pl.ds(r * ch + k * 16, 16)] = accs_[k]
                r2 = r + 1
                row_end2 = rp_v[pl.ds((r2 + 1) * 16, 16)][0]
                return (r2, row_end2, zed,
                        tuple(zed for _ in range(nacc)))

            def keep(den_, accs_):
                return (r, row_end, den_, accs_)

            return lax.cond(is_end, flush, keep, den, tuple(naccs))

        def super_body(s, carry):
            sb = base0 + s * ECAP
            pltpu.sync_copy(srcp_hbm.at[pl.ds(sb, ECAP)], srcall_v)
            cnt = jnp.minimum(ECAP, e1 - sb)
            nchs = (cnt + CG - 1) // CG

            def issue(coff, buf, sm):
                pltpu.async_copy(
                    hrow_hbm.at[srcall_v.at[pl.ds(coff, CG)]], buf, sm)

            def drain(buf, sm):
                pltpu.make_async_copy(
                    hrow_hbm.at[pl.ds(0, CG)], buf, sm).wait()

            def process(carry, coff, buf, masked):
                gbase = sb + coff

                def group(g, cr):
                    for i in range(16):
                        cr = per_edge(masked, gbase, g * 16 + i, cr, buf)
                    return cr

                return lax.fori_loop(0, CG // 16, group, carry)

            def proc_chunk(carry, coff, masked):
                issue(coff, gbuf_v, sem)
                drain(gbuf_v, sem)
                return process(carry, coff, gbuf_v, masked)

            carry = proc_chunk(carry, 0, True)
            hi = nchs - 1

            @pl.when(hi > 1)
            def _prologue():
                issue(1 * CG, gbuf_v, sem)

            def pair(p, cr):
                c0 = 1 + 2 * p

                @pl.when(c0 + 1 < hi)
                def _i1():
                    issue((c0 + 1) * CG, gbuf2_v, sem2)

                drain(gbuf_v, sem)
                cr = process(cr, c0 * CG, gbuf_v, False)

                @pl.when(c0 + 2 < hi)
                def _i2():
                    issue((c0 + 2) * CG, gbuf_v, sem)

                def second(c, cr2):
                    drain(gbuf2_v, sem2)
                    return process(cr2, c * CG, gbuf2_v, False)

                return lax.fori_loop(
                    c0 + 1, jnp.minimum(c0 + 2, hi), second, cr)

            m = jnp.maximum(hi - 1, 0)
            carry = lax.fori_loop(0, m // 2 + m % 2, pair, carry)
            carry = lax.fori_loop(
                jnp.maximum(nchs - 1, 1), nchs,
                lambda c, cr: proc_chunk(cr, c * CG, True), carry)
            return carry

        init = (jnp.int32(0), rp_v[pl.ds(16, 16)][0],
                zed, tuple(zed for _ in range(nacc)))
        lax.fori_loop(0, nsuper, super_body, init)
        pltpu.sync_copy(out_v, out_hbm.at[pl.ds(d0 * ch, ROWS * ch)])
        pltpu.sync_copy(den_v, deno_hbm.at[pl.ds(d0 * 16, ROWS * 16)])

    return gat_sc


_gat_sc_256 = _make_gat_sc(256)
_gat_sc_128 = _make_gat_sc(128)


def _make_appnp_sc():
    mesh = plsc.VectorSubcoreMesh(core_axis_name="c", subcore_axis_name="s")

    @functools.partial(
        pl.kernel,
        mesh=mesh,
        out_type=[
            jax.ShapeDtypeStruct((NP_ * H2,), jnp.float32),
            jax.ShapeDtypeStruct((NP_ * H2,), jnp.float32),
        ],
        compiler_params=pltpu.CompilerParams(use_tc_tiling_on_sc=False),
        scratch_types=[
            pltpu.VMEM(((ROWS + 16) * 16,), jnp.int32),
            pltpu.VMEM((ROWS * H2,), jnp.float32),
            pltpu.VMEM((ROWS * 16,), jnp.float32),
            pltpu.VMEM((ECAP,), jnp.int32),
            pltpu.VMEM((ECAP,), jnp.int32),
            pltpu.VMEM((CA, H2), jnp.float32),
            pltpu.VMEM((CA, H2), jnp.float32),
            pltpu.VMEM((ROWS * H2,), jnp.float32),
            pltpu.VMEM((ROWS * H2,), jnp.float32),
            pltpu.VMEM((ROWS * H2,), jnp.float32),
            pltpu.SemaphoreType.DMA,
            pltpu.SemaphoreType.DMA,
        ],
    )
    def appnp_sc(y_hbm, h_hbm, dis_hbm, rp_hbm, srcp_hbm, dstp_hbm,
                 z_hbm, yo_hbm,
                 rp_v, h_v, dis_v, srcall_v, dstall_v, ybuf_v, ybuf2_v,
                 acc_v, z_v, yo_v, sem, sem2):
        zed = jnp.zeros((16,), jnp.float32)
        wid = lax.axis_index("s") * 2 + lax.axis_index("c")
        d0 = wid * ROWS
        pltpu.sync_copy(rp_hbm.at[pl.ds(d0 * 16, (ROWS + 16) * 16)], rp_v)
        pltpu.sync_copy(h_hbm.at[pl.ds(d0 * H2, ROWS * H2)], h_v)
        pltpu.sync_copy(dis_hbm.at[pl.ds(d0 * 16, ROWS * 16)], dis_v)
        e0 = rp_v[pl.ds(0, 16)][0]
        e1 = rp_v[pl.ds(ROWS * 16, 16)][0]
        base0 = (e0 // CA) * CA
        nsuper = (e1 - base0 + ECAP - 1) // ECAP

        def zero_row(r, _):
            acc_v[pl.ds(r * H2, 16)] = zed
            acc_v[pl.ds(r * H2 + 16, 16)] = zed
            return 0

        lax.fori_loop(0, ROWS, zero_row, 0)

        def per_edge(masked, gbase, gi, dvec, lane, buf):
            dl = dvec[lane] - d0
            y0 = buf[gi, pl.ds(0, 16)]
            y1 = buf[gi, pl.ds(16, 16)]
            if masked:
                gidx = gbase + gi
                vm = jnp.logical_and(gidx >= e0, gidx < e1)
                vmv = jnp.broadcast_to(
                    jnp.where(vm, jnp.float32(1.0), jnp.float32(0.0)), (16,))
                y0 = y0 * vmv
                y1 = y1 * vmv
                dl = jnp.where(vm, dl, 0)
            plsc.addupdate(acc_v.at[pl.ds(dl * H2, 16)], y0)
            plsc.addupdate(acc_v.at[pl.ds(dl * H2 + 16, 16)], y1)

        def super_body(s, _):
            sb = base0 + s * ECAP
            pltpu.sync_copy(srcp_hbm.at[pl.ds(sb, ECAP)], srcall_v)
            pltpu.sync_copy(dstp_hbm.at[pl.ds(sb, ECAP)], dstall_v)
            cnt = jnp.minimum(ECAP, e1 - sb)
            nchs = (cnt + CA - 1) // CA

            def issue(coff, buf, sm):
                pltpu.async_copy(
                    y_hbm.at[srcall_v.at[pl.ds(coff, CA)]], buf, sm)

            def drain(buf, sm):
                pltpu.make_async_copy(y_hbm.at[pl.ds(0, CA)], buf, sm).wait()

            def process(coff, buf, masked):
                gbase = sb + coff

                def group(g, _):
                    dvec = dstall_v[pl.ds(coff + g * 16, 16)]
                    for i in range(16):
                        per_edge(masked, gbase, g * 16 + i, dvec, i,
                                 buf)
                    return 0

                lax.fori_loop(0, CA // 16, group, 0)

            def proc_chunk(coff, masked):
                issue(coff, ybuf_v, sem)
                drain(ybuf_v, sem)
                process(coff, ybuf_v, masked)

            proc_chunk(0, True)
            hi = nchs - 1

            @pl.when(hi > 1)
            def _prologue():
                issue(1 * CA, ybuf_v, sem)

            def pair(p, _):
                c0 = 1 + 2 * p

                @pl.when(c0 + 1 < hi)
                def _i1():
                    issue((c0 + 1) * CA, ybuf2_v, sem2)

                drain(ybuf_v, sem)
                process(c0 * CA, ybuf_v, False)

                @pl.when(c0 + 2 < hi)
                def _i2():
                    issue((c0 + 2) * CA, ybuf_v, sem)

                @pl.when(c0 + 1 < hi)
                def _p2():
                    drain(ybuf2_v, sem2)
                    process((c0 + 1) * CA, ybuf2_v, False)

                return 0

            npair = jnp.maximum(hi - 1, 0) // 2 + jnp.maximum(hi - 1, 0) % 2
            lax.fori_loop(0, npair, pair, 0)
            lax.fori_loop(
                jnp.maximum(nchs - 1, 1), nchs,
                lambda c, cr: (proc_chunk(c * CA, True), 0)[1], 0)
            return 0

        lax.fori_loop(0, nsuper, super_body, 0)

        def post_row(r, _):
            sv = jnp.broadcast_to(dis_v[pl.ds(r * 16, 16)][0], (16,))
            a0 = acc_v[pl.ds(r * H2, 16)]
            a1 = acc_v[pl.ds(r * H2 + 16, 16)]
            z0 = (0.9 * sv) * a0 + 0.1 * h_v[pl.ds(r * H2, 16)]
            z1 = (0.9 * sv) * a1 + 0.1 * h_v[pl.ds(r * H2 + 16, 16)]
            z_v[pl.ds(r * H2, 16)] = z0
            z_v[pl.ds(r * H2 + 16, 16)] = z1
            yo_v[pl.ds(r * H2, 16)] = sv * z0
            yo_v[pl.ds(r * H2 + 16, 16)] = sv * z1
            return 0

        lax.fori_loop(0, ROWS, post_row, 0)
        pltpu.sync_copy(z_v, z_hbm.at[pl.ds(d0 * H2, ROWS * H2)])
        pltpu.sync_copy(yo_v, yo_hbm.at[pl.ds(d0 * H2, ROWS * H2)])

    return appnp_sc


_appnp_sc = _make_appnp_sc()


def _pool_final_body(z_ref, aux4_ref, batch_ref, gbc_ref,
                     wgz1_ref, wga1_ref, bg11_ref, wg12_ref, bg12_ref,
                     wnz1_ref, wna1_ref, bn11_ref, wn12_ref, bn12_ref,
                     wgz2_ref, wga2_ref, bg21_ref, wg22_ref, bg22_ref,
                     wnz2_ref, wna2_ref, bn21_ref, wn22_ref, bn22_ref,
                     wgz3_ref, wga3_ref, bg31_ref, wg32_ref, bg32_ref,
                     wnz3_ref, wna3_ref, bn31_ref, wn32_ref, bn32_ref,
                     w3z_ref, w3g_ref, w3a_ref, w3b_ref, b3_ref,
                     w4_ref, b4_ref, w5_ref, b5_ref,
                     out_ref):
    z = z_ref[...]
    aux4 = aux4_ref[...]
    aux3 = aux4[:, 0:3]
    batch = batch_ref[...]
    iota8 = lax.broadcasted_iota(jnp.int32, (1, B), 1)
    mask = (batch == iota8).astype(jnp.float32)

    def pool(wgz, wga, bg1, wg2, bg2, wnz, wna, bn1, wn2, bn2):
        gate = jnp.maximum(z @ wgz + aux3 @ wga + bg1, 0.0) @ wg2 + bg2
        v = jnp.maximum(z @ wnz + aux3 @ wna + bn1, 0.0) @ wn2 + bn2
        gm = jnp.where(mask > 0, gate, -1e30)
        m = jnp.max(gm, axis=0)
        m = jnp.where(m > -1e29, m, 0.0)
        ex = jnp.exp(gate - mask @ m[:, None])
        den = lax.dot_general(mask, ex, (((0,), (0,)), ((), ())))
        a = ex / (mask @ den + 1e-16)
        return lax.dot_general(mask, a * v, (((0,), (0,)), ((), ())))

    g1 = pool(wgz1_ref[...], wga1_ref[...], bg11_ref[...], wg12_ref[...],
              bg12_ref[...], wnz1_ref[...], wna1_ref[...], bn11_ref[...],
              wn12_ref[...], bn12_ref[...])
    g2 = pool(wgz2_ref[...], wga2_ref[...], bg21_ref[...], wg22_ref[...],
              bg22_ref[...], wnz2_ref[...], wna2_ref[...], bn21_ref[...],
              wn22_ref[...], bn22_ref[...])
    g3 = pool(wgz3_ref[...], wga3_ref[...], bg31_ref[...], wg32_ref[...],
              bg32_ref[...], wnz3_ref[...], wna3_ref[...], bn31_ref[...],
              wn32_ref[...], bn32_ref[...])
    g = jnp.concatenate([g1, g2, g3], axis=1)
    tg = g @ w3g_ref[...] + gbc_ref[...] @ w3b_ref[...]
    s = _lrelu(z @ w3z_ref[...] + aux4 @ w3a_ref[...] + mask @ tg
               + b3_ref[...])
    s = _lrelu(s @ w4_ref[...] + b4_ref[...])
    s = s @ w5_ref[...] + b5_ref[...]
    s = jax.nn.sigmoid(s)
    out_ref[...] = lax.dot_general(mask, s, (((0,), (0,)), ((), ())))


def _pool_final(z, aux4, batch, gbc, pool_ws, w3z, w3g, w3a, w3b, b3,
                w4, b4, w5, b5):
    flat = []
    for ws in pool_ws:
        flat.extend(ws)
    return pl.pallas_call(
        _pool_final_body,
        out_shape=jax.ShapeDtypeStruct((B, 1), jnp.float32),
    )(z, aux4, batch, gbc, *flat, w3z, w3g, w3a, w3b, b3, w4, b4, w5, b5)


def _block_diag_att(att_src, att_dst):
    heads, ch = att_src.shape
    eye = jnp.eye(heads, dtype=att_src.dtype)
    a_s = jnp.einsum("hc,hk->hck", att_src, eye).reshape(heads * ch, heads)
    a_d = jnp.einsum("hc,hk->hck", att_dst, eye).reshape(heads * ch, heads)
    return jnp.concatenate([a_s, a_d], axis=1)


def kernel(x, edge_index, batch, J, saved_nodes, infected_nodes,
           size_connected, Omegas, Phis, Lambdas,
           W_gat1, att_src1, att_dst1, b_gat1, W_lin1, b_lin1,
           W_gat2, att_src2, att_dst2, b_gat2, W_lin2, b_lin2,
           Wg11, bg11, Wg12, bg12, Wn11, bn11, Wn12, bn12,
           Wg21, bg21, Wg22, bg22, Wn21, bn21, Wn22, bn22,
           Wg31, bg31, Wg32, bg32, Wn31, bn31, Wn32, bn32,
           W_lin3, b_lin3, W_lin4, b_lin4, W_lin5, b_lin5):
    idt = edge_index.dtype
    loop = jnp.arange(N, dtype=idt)
    src = jnp.concatenate([edge_index[0], loop])
    dst = jnp.concatenate([edge_index[1], loop])
    dst_s, src_s = lax.sort((dst, src), num_keys=1)

    fake_dst = jnp.arange(N, NP_, dtype=idt)
    dst_f = jnp.concatenate([dst_s, fake_dst])
    src_f = jnp.concatenate([src_s, jnp.zeros((NP_ - N,), idt)])
    npad = EP - src_f.shape[0]
    srcp = jnp.concatenate([src_f, jnp.zeros((npad,), idt)])
    dstp = jnp.concatenate([dst_f, jnp.zeros((npad,), idt)])

    rpf = jnp.searchsorted(
        dst_f, jnp.arange(NP_ + 16, dtype=idt)).astype(jnp.int32)
    rp_rep = jnp.repeat(rpf, 16)
    rp = jnp.searchsorted(dst_s, jnp.arange(N + 1, dtype=idt))
    deg = (rp[1:] - rp[:-1]).astype(jnp.float32)[:, None]
    deg = jnp.concatenate([deg, jnp.zeros((NP_ - N, 1), jnp.float32)])

    zrows = jnp.zeros((NP_ - N, 1), jnp.float32)
    x_p = jnp.concatenate([x, jnp.zeros((NP_ - N, IN_DIM), jnp.float32)])
    aux_p = jnp.concatenate([
        jnp.concatenate([J, zrows]), jnp.concatenate([size_connected, zrows]),
        jnp.zeros((NP_, 6), jnp.float32)], axis=1)

    wx = W_gat1[:IN_DIM]
    wjs = jnp.concatenate(
        [W_gat1[IN_DIM:], jnp.zeros((6, HEADS * H1), jnp.float32)])
    asd1 = _block_diag_att(att_src1, att_dst1)
    asd2 = _block_diag_att(att_src2, att_dst2)

    eye4 = jnp.eye(HEADS, dtype=jnp.float32)
    r1 = jnp.repeat(eye4, H1, axis=1)
    r2 = jnp.repeat(eye4, H2, axis=1)

    hrow1, edp1 = _dense1(x_p, aux_p, wx, wjs, asd1)
    out1, den1 = _gat_sc_256(hrow1, edp1.reshape(-1), rp_rep, srcp, dstp)
    hrow2, edp2 = _dense2(out1.reshape(NP_, 256), den1.reshape(NP_, 16), r1,
                          b_gat1, W_lin1, b_lin1, W_gat2, asd2)
    out2, den2 = _gat_sc_128(hrow2, edp2.reshape(-1), rp_rep, srcp, dstp)
    h32, dis16, y = _dense3(out2.reshape(NP_, 128), den2.reshape(NP_, 16), r2,
                            deg, b_gat2, W_lin2, b_lin2)

    h32f = h32.reshape(-1)
    dis16f = dis16.reshape(-1)
    z = None
    for _ in range(K_POWER):
        zf, yf = _appnp_sc(y, h32f, dis16f, rp_rep, srcp, dstp)
        y = yf.reshape(NP_, H2)
        z = zf.reshape(NP_, H2)

    aux4 = jnp.concatenate([J, saved_nodes, infected_nodes, size_connected],
                           axis=1)
    gbc = jnp.concatenate([Omegas, Phis, Lambdas], axis=1)
    pool_ws = [
        (Wg11[:H2], Wg11[H2:], bg11, Wg12, bg12,
         Wn11[:H2], Wn11[H2:], bn11, Wn12, bn12),
        (Wg21[:H2], Wg21[H2:], bg21, Wg22, bg22,
         Wn21[:H2], Wn21[H2:], bn21, Wn22, bn22),
        (Wg31[:H2], Wg31[H2:], bg31, Wg32, bg32,
         Wn31[:H2], Wn31[H2:], bn31, Wn32, bn32),
    ]
    w3z = W_lin3[0:H2]
    w3g = W_lin3[H2:H2 + 3 * H2]
    w3a = W_lin3[H2 + 3 * H2:H2 + 3 * H2 + 4]
    w3b = W_lin3[H2 + 3 * H2 + 4:]
    return _pool_final(z[:N], aux4, batch[:, None], gbc, pool_ws,
                       w3z, w3g, w3a, w3b, b_lin3,
                       W_lin4, b_lin4, W_lin5, b_lin5)
